# Initial kernel scaffold; baseline (speedup 1.0000x reference)
import jax
import jax.numpy as jnp
from jax.experimental import pallas as pl


def kernel(x, edge_index, edge_type, batch, W1, q1, k1, b1, W2, q2, k2, b2, lin1_w, lin1_b, lin2_w, lin2_b):
    raise NotImplementedError("write your pallas kernel here")


# speedup vs baseline: 7.9254x; 7.9254x over previous
import functools

import jax
import jax.numpy as jnp
from jax import lax
from jax.experimental import pallas as pl
from jax.experimental.pallas import tpu as pltpu

N = 10000
E = 320000
R = 8
H = 4
D = 64
G = 64
HD = H * D

NP = 10240
BM = 512
BN = 256


def _mm_body(x_ref, w_ref, q_ref, k_ref, b_ref, h_ref, aq_ref, ak_ref, *, act):
    xb = x_ref[...]
    if act:
        xb = xb + b_ref[...]
        xb = jnp.where(xb > 0, xb, jnp.exp(jnp.minimum(xb, 0.0)) - 1.0)
    h = jnp.dot(xb, w_ref[0], preferred_element_type=jnp.float32)
    h4 = h.reshape(BM, H, D)
    aq = jnp.sum(h4 * q_ref[0][None], axis=-1)
    ak = jnp.sum(h4 * k_ref[0][None], axis=-1)
    h_ref[0] = h
    aq_ref[0] = aq
    ak_ref[0] = ak


def _relational_transform(x_pad, W, q_att, k_att, bias2d, act):
    F = x_pad.shape[1]
    grid = (R, NP // BM)
    out_shapes = (
        jax.ShapeDtypeStruct((R, NP, HD), jnp.float32),
        jax.ShapeDtypeStruct((R, NP, H), jnp.float32),
        jax.ShapeDtypeStruct((R, NP, H), jnp.float32),
    )
    return pl.pallas_call(
        functools.partial(_mm_body, act=act),
        grid=grid,
        in_specs=[
            pl.BlockSpec((BM, F), lambda r, i: (i, 0)),
            pl.BlockSpec((1, F, HD), lambda r, i: (r, 0, 0)),
            pl.BlockSpec((1, H, D), lambda r, i: (r, 0, 0)),
            pl.BlockSpec((1, H, D), lambda r, i: (r, 0, 0)),
            pl.BlockSpec((1, HD), lambda r, i: (0, 0)),
        ],
        out_specs=(
            pl.BlockSpec((1, BM, HD), lambda r, i: (r, i, 0)),
            pl.BlockSpec((1, BM, H), lambda r, i: (r, i, 0)),
            pl.BlockSpec((1, BM, H), lambda r, i: (r, i, 0)),
        ),
        out_shape=out_shapes,
    )(x_pad, W, q_att, k_att, bias2d)


def _pool_body(z_ref, b2_ref, batch_ref, bcol_ref, gr_ref, l1w_ref, l1b_ref,
               l2w_ref, l2b_ref, out_ref, acc_s, acc_c, acc_m):
    i = pl.program_id(0)
    nb = pl.num_programs(0)

    @pl.when(i == 0)
    def _():
        acc_s[...] = jnp.zeros((G, D), jnp.float32)
        acc_c[...] = jnp.zeros((G, D), jnp.float32)
        acc_m[...] = jnp.full((G, D), -jnp.inf, jnp.float32)

    zb = z_ref[...] + b2_ref[...]
    xb = jnp.where(zb > 0, zb, jnp.exp(jnp.minimum(zb, 0.0)) - 1.0)
    bb = batch_ref[0]
    gids = lax.broadcasted_iota(jnp.int32, (G, BN), 0)
    mask = bb == gids
    maskf = mask.astype(jnp.float32)
    acc_s[...] += jnp.dot(maskf, xb, preferred_element_type=jnp.float32)
    acc_c[...] += jnp.dot(maskf, jnp.ones((BN, D), jnp.float32),
                          preferred_element_type=jnp.float32)
    bcol = bcol_ref[0]
    g_lo = gr_ref[i, 0]
    g_hi = gr_ref[i, 1]

    def _max_body(g, carry):
        m = jnp.where(bcol == g, xb, -jnp.inf)
        mx_g = jnp.max(m, axis=0, keepdims=True)
        acc_m[pl.ds(g, 1), :] = jnp.maximum(acc_m[pl.ds(g, 1), :], mx_g)
        return carry

    lax.fori_loop(g_lo, g_hi + 1, _max_body, 0)

    @pl.when(i == nb - 1)
    def _():
        mean = acc_s[...] / jnp.maximum(acc_c[...], 1.0)
        mxv = acc_m[...]
        mxv = jnp.where(mxv == -jnp.inf, 0.0, mxv)
        pool = mean + mxv
        h = jnp.maximum(
            jnp.dot(pool, l1w_ref[...], preferred_element_type=jnp.float32)
            + l1b_ref[...], 0.0)
        out_ref[...] = (
            jnp.dot(h, l2w_ref[...], preferred_element_type=jnp.float32)
            + l2b_ref[...])


def _edge_aggregate(h_all, aq_node, ak_node, src, dst, et):
    aqf = aq_node.reshape(R * NP, H)
    akf = ak_node.reshape(R * NP, H)
    aq = aqf[et * NP + dst]
    ak = akf[et * NP + src]
    e = aq + ak
    e = jnp.where(e >= 0, e, 0.2 * e)
    ex = jnp.exp(e)
    denom = jax.ops.segment_sum(ex, dst, num_segments=NP)
    alpha = ex / denom[dst]
    h_src = h_all.reshape(R * NP, HD)[et * NP + src].reshape(E, H, D)
    msg = (h_src * alpha[:, :, None]).reshape(E, HD)
    return jax.ops.segment_sum(msg, dst, num_segments=NP)


def kernel(x, edge_index, edge_type, batch, W1, q1, k1, b1, W2, q2, k2, b2,
           lin1_w, lin1_b, lin2_w, lin2_b):
    src = edge_index[0]
    dst = edge_index[1]
    et = edge_type

    x_pad = jnp.pad(x, ((0, NP - N), (0, 0)))
    b1_2d = b1.reshape(1, HD)
    b2_2d = b2.reshape(1, D)

    h1, aq1, ak1 = _relational_transform(x_pad, W1, q1, k1, b1_2d, act=False)
    raw1 = _edge_aggregate(h1, aq1, ak1, src, dst, et)

    h2, aq2, ak2 = _relational_transform(raw1, W2, q2, k2, b1_2d, act=True)
    raw2 = _edge_aggregate(h2, aq2, ak2, src, dst, et)
    z = raw2.reshape(NP, H, D).mean(axis=1)

    batch_pad = jnp.pad(batch, (0, NP - N), constant_values=G)
    batch3 = batch_pad.reshape(NP // BN, 1, BN)
    batch_col = batch_pad.reshape(NP // BN, BN, 1)
    rows = batch_pad.reshape(NP // BN, BN)
    grange = jnp.stack(
        [rows[:, 0], jnp.minimum(rows[:, -1], G - 1)], axis=1
    ).astype(jnp.int32)
    out = pl.pallas_call(
        _pool_body,
        grid=(NP // BN,),
        in_specs=[
            pl.BlockSpec((BN, D), lambda i: (i, 0)),
            pl.BlockSpec((1, D), lambda i: (0, 0)),
            pl.BlockSpec((1, 1, BN), lambda i: (i, 0, 0)),
            pl.BlockSpec((1, BN, 1), lambda i: (i, 0, 0)),
            pl.BlockSpec(memory_space=pltpu.SMEM),
            pl.BlockSpec((D, D), lambda i: (0, 0)),
            pl.BlockSpec((1, D), lambda i: (0, 0)),
            pl.BlockSpec((D, G), lambda i: (0, 0)),
            pl.BlockSpec((1, G), lambda i: (0, 0)),
        ],
        out_specs=pl.BlockSpec((G, G), lambda i: (0, 0)),
        out_shape=jax.ShapeDtypeStruct((G, G), jnp.float32),
        scratch_shapes=[
            pltpu.VMEM((G, D), jnp.float32),
            pltpu.VMEM((G, D), jnp.float32),
            pltpu.VMEM((G, D), jnp.float32),
        ],
    )(z, b2_2d, batch3, batch_col, grange, lin1_w, lin1_b.reshape(1, D),
      lin2_w, lin2_b.reshape(1, G))
    return out


# speedup vs baseline: 23.0888x; 2.9133x over previous
import functools

import jax
import jax.numpy as jnp
from jax import lax
from jax.experimental import pallas as pl
from jax.experimental.pallas import tpu as pltpu
from jax.experimental.pallas import tpu_sc as plsc

N = 10000
E = 320000
R = 8
H = 4
D = 64
G = 64
HD = H * D

NP = 10240
BM = 512
BN = 256


def _mm_body(x_ref, w_ref, q_ref, k_ref, b_ref, h_ref, aq_ref, ak_ref, *, act):
    xb = x_ref[...]
    if act:
        xb = xb + b_ref[...]
        xb = jnp.where(xb > 0, xb, jnp.exp(jnp.minimum(xb, 0.0)) - 1.0)
    h = jnp.dot(xb, w_ref[0], preferred_element_type=jnp.float32)
    h4 = h.reshape(BM, H, D)
    aq = jnp.sum(h4 * q_ref[0][None], axis=-1)
    ak = jnp.sum(h4 * k_ref[0][None], axis=-1)
    h_ref[0] = h
    aq_ref[0] = aq
    ak_ref[0] = ak


def _relational_transform(x_pad, W, q_att, k_att, bias2d, act):
    F = x_pad.shape[1]
    grid = (R, NP // BM)
    out_shapes = (
        jax.ShapeDtypeStruct((R, NP, HD), jnp.float32),
        jax.ShapeDtypeStruct((R, NP, H), jnp.float32),
        jax.ShapeDtypeStruct((R, NP, H), jnp.float32),
    )
    return pl.pallas_call(
        functools.partial(_mm_body, act=act),
        grid=grid,
        in_specs=[
            pl.BlockSpec((BM, F), lambda r, i: (i, 0)),
            pl.BlockSpec((1, F, HD), lambda r, i: (r, 0, 0)),
            pl.BlockSpec((1, H, D), lambda r, i: (r, 0, 0)),
            pl.BlockSpec((1, H, D), lambda r, i: (r, 0, 0)),
            pl.BlockSpec((1, HD), lambda r, i: (0, 0)),
        ],
        out_specs=(
            pl.BlockSpec((1, BM, HD), lambda r, i: (r, i, 0)),
            pl.BlockSpec((1, BM, H), lambda r, i: (r, i, 0)),
            pl.BlockSpec((1, BM, H), lambda r, i: (r, i, 0)),
        ),
        out_shape=out_shapes,
    )(x_pad, W, q_att, k_att, bias2d)


def _mm_body2(a_ref, b_ref, w_ref, q_ref, k_ref, bias_ref,
              h_ref, aq_ref, ak_ref):
    bias = bias_ref[...]
    xa = a_ref[...] + bias[:, 0:128]
    xa = jnp.where(xa > 0, xa, jnp.exp(jnp.minimum(xa, 0.0)) - 1.0)
    xb = b_ref[...] + bias[:, 128:256]
    xb = jnp.where(xb > 0, xb, jnp.exp(jnp.minimum(xb, 0.0)) - 1.0)
    w = w_ref[0]
    h = (jnp.dot(xa, w[0:128], preferred_element_type=jnp.float32)
         + jnp.dot(xb, w[128:256], preferred_element_type=jnp.float32))
    h4 = h.reshape(BM, H, D)
    aq_ref[0] = jnp.sum(h4 * q_ref[0][None], axis=-1)
    ak_ref[0] = jnp.sum(h4 * k_ref[0][None], axis=-1)
    h_ref[0] = h


def _relational_transform2(a, b, W, q_att, k_att, bias2d):
    grid = (R, NP // BM)
    out_shapes = (
        jax.ShapeDtypeStruct((R, NP, HD), jnp.float32),
        jax.ShapeDtypeStruct((R, NP, H), jnp.float32),
        jax.ShapeDtypeStruct((R, NP, H), jnp.float32),
    )
    return pl.pallas_call(
        _mm_body2,
        grid=grid,
        in_specs=[
            pl.BlockSpec((BM, 128), lambda r, i: (i, 0)),
            pl.BlockSpec((BM, 128), lambda r, i: (i, 0)),
            pl.BlockSpec((1, HD, HD), lambda r, i: (r, 0, 0)),
            pl.BlockSpec((1, H, D), lambda r, i: (r, 0, 0)),
            pl.BlockSpec((1, H, D), lambda r, i: (r, 0, 0)),
            pl.BlockSpec((1, HD), lambda r, i: (0, 0)),
        ],
        out_specs=(
            pl.BlockSpec((1, BM, HD), lambda r, i: (r, i, 0)),
            pl.BlockSpec((1, BM, H), lambda r, i: (r, i, 0)),
            pl.BlockSpec((1, BM, H), lambda r, i: (r, i, 0)),
        ),
        out_shape=out_shapes,
    )(a, b, W, q_att, k_att, bias2d)


def _pool_body(pa_ref, pb_ref, b2_ref, batch_ref, bcol_ref, gr_ref, l1w_ref,
               l1b_ref, l2w_ref, l2b_ref, out_ref, acc_s, acc_c, acc_m):
    i = pl.program_id(0)
    nb = pl.num_programs(0)

    @pl.when(i == 0)
    def _():
        acc_s[...] = jnp.zeros((G, D), jnp.float32)
        acc_c[...] = jnp.zeros((G, D), jnp.float32)
        acc_m[...] = jnp.full((G, D), -jnp.inf, jnp.float32)

    pa = pa_ref[...]
    pb = pb_ref[...]
    zb = (pa[:, 0:64] + pa[:, 64:128] + pb[:, 0:64] + pb[:, 64:128]) * 0.25
    zb = zb + b2_ref[...]
    xb = jnp.where(zb > 0, zb, jnp.exp(jnp.minimum(zb, 0.0)) - 1.0)
    bb = batch_ref[0]
    gids = lax.broadcasted_iota(jnp.int32, (G, BN), 0)
    mask = bb == gids
    maskf = mask.astype(jnp.float32)
    acc_s[...] += jnp.dot(maskf, xb, preferred_element_type=jnp.float32)
    acc_c[...] += jnp.dot(maskf, jnp.ones((BN, D), jnp.float32),
                          preferred_element_type=jnp.float32)
    bcol = bcol_ref[0]
    g_lo = gr_ref[i, 0]
    g_hi = gr_ref[i, 1]

    def _max_body(g, carry):
        m = jnp.where(bcol == g, xb, -jnp.inf)
        mx_g = jnp.max(m, axis=0, keepdims=True)
        acc_m[pl.ds(g, 1), :] = jnp.maximum(acc_m[pl.ds(g, 1), :], mx_g)
        return carry

    lax.fori_loop(g_lo, g_hi + 1, _max_body, 0)

    @pl.when(i == nb - 1)
    def _():
        mean = acc_s[...] / jnp.maximum(acc_c[...], 1.0)
        mxv = acc_m[...]
        mxv = jnp.where(mxv == -jnp.inf, 0.0, mxv)
        pool = mean + mxv
        h = jnp.maximum(
            jnp.dot(pool, l1w_ref[...], preferred_element_type=jnp.float32)
            + l1b_ref[...], 0.0)
        out_ref[...] = (
            jnp.dot(h, l2w_ref[...], preferred_element_type=jnp.float32)
            + l2b_ref[...])


CB = 80
NTILES = 32
EPW_A = E // NTILES
EPW_B = E // 16
STRIPE = NP // 16

_MESH = plsc.VectorSubcoreMesh(core_axis_name="c", subcore_axis_name="s")
_SC_PARAMS = pltpu.CompilerParams(use_tc_tiling_on_sc=False)


def _lanes():
    return lax.iota(jnp.int32, 16)


HP = 16


def _edge_attention_sc(aqf, akf, src, dst, et, zeros_h):
    @functools.partial(
        pl.kernel,
        mesh=_MESH,
        compiler_params=_SC_PARAMS,
        out_type=(
            jax.ShapeDtypeStruct((E, HP), jnp.float32),
            jax.ShapeDtypeStruct((2, NP, HP), jnp.float32),
        ),
        scratch_types=[
            pltpu.VMEM((CB,), jnp.int32),
            pltpu.VMEM((CB,), jnp.int32),
            pltpu.VMEM((CB,), jnp.int32),
            pltpu.VMEM((CB,), jnp.int32),
            pltpu.VMEM((CB,), jnp.int32),
            pltpu.VMEM((CB, HP), jnp.float32),
            pltpu.VMEM((CB, HP), jnp.float32),
            pltpu.VMEM((CB, HP), jnp.float32),
            pltpu.VMEM_SHARED((NP, HP), jnp.float32),
            pltpu.SemaphoreType.DMA,
            pltpu.SemaphoreType.DMA,
        ],
    )
    def k(aq_hbm, ak_hbm, src_hbm, dst_hbm, et_hbm, z_hbm,
          ex_hbm, den_hbm,
          src_v, dst_v, et_v, idxq_v, idxk_v, aq_v, ak_v, ex_v,
          den_sp, sem0, sem1):
        cid = lax.axis_index("c")
        sid = lax.axis_index("s")
        wid = sid * 2 + cid

        pltpu.sync_copy(z_hbm.at[pl.ds(sid * STRIPE, STRIPE), :],
                        den_sp.at[pl.ds(sid * STRIPE, STRIPE), :])
        plsc.subcore_barrier()

        def chunk(t, carry):
            base = wid * EPW_A + t * CB
            pltpu.sync_copy(src_hbm.at[pl.ds(base, CB)], src_v)
            pltpu.sync_copy(dst_hbm.at[pl.ds(base, CB)], dst_v)
            pltpu.sync_copy(et_hbm.at[pl.ds(base, CB)], et_v)
            for j in range(CB // 16):
                sl = pl.ds(j * 16, 16)
                etj = et_v[sl] * NP
                idxq_v[sl] = etj + dst_v[sl]
                idxk_v[sl] = etj + src_v[sl]
            cp0 = pltpu.async_copy(aq_hbm.at[idxq_v], aq_v, sem0)
            cp1 = pltpu.async_copy(ak_hbm.at[idxk_v], ak_v, sem1)
            cp0.wait()
            cp1.wait()
            for r in range(CB):
                ev = aq_v[r, :] + ak_v[r, :]
                ev = jnp.where(ev >= 0, ev, 0.2 * ev)
                ex_v[r, :] = jnp.exp(ev)
            pltpu.sync_copy(ex_v, ex_hbm.at[pl.ds(base, CB), :])
            pltpu.sync_copy(ex_v, den_sp.at[dst_v], add=True)
            return carry

        lax.fori_loop(0, EPW_A // CB, chunk, 0)
        plsc.subcore_barrier()
        pltpu.sync_copy(den_sp.at[pl.ds(sid * STRIPE, STRIPE), :],
                        den_hbm.at[cid, pl.ds(sid * STRIPE, STRIPE), :])

    return k(aqf, akf, src, dst, et, zeros_h)


def _edge_apply_sc(h_flat, ex, den_a, den_b, src, dst, et, zeros_d):
    @functools.partial(
        pl.kernel,
        mesh=_MESH,
        compiler_params=_SC_PARAMS,
        out_type=jax.ShapeDtypeStruct((2, NP, 128), jnp.float32),
        scratch_types=[
            pltpu.VMEM((CB,), jnp.int32),
            pltpu.VMEM((CB,), jnp.int32),
            pltpu.VMEM((CB,), jnp.int32),
            pltpu.VMEM((CB,), jnp.int32),
            pltpu.VMEM((CB, HP), jnp.float32),
            pltpu.VMEM((CB, HP), jnp.float32),
            pltpu.VMEM((CB, HP), jnp.float32),
            pltpu.VMEM((CB, 128), jnp.float32),
            pltpu.VMEM_SHARED((NP, 128), jnp.float32),
            pltpu.SemaphoreType.DMA,
            pltpu.SemaphoreType.DMA,
            pltpu.SemaphoreType.DMA,
        ],
    )
    def k(h_hbm, ex_hbm, dena_hbm, denb_hbm, src_hbm, dst_hbm, et_hbm, z_hbm,
          out_hbm,
          src_v, dst_v, et_v, idx_v, ex_v, da_v, db_v, rows_v,
          acc_sp, sem0, sem1, sem2):
        cid = lax.axis_index("c")
        sid = lax.axis_index("s")

        pltpu.sync_copy(z_hbm.at[pl.ds(sid * STRIPE, STRIPE), :],
                        acc_sp.at[pl.ds(sid * STRIPE, STRIPE), :])
        plsc.subcore_barrier()

        def chunk(t, carry, hh):
            base = sid * EPW_B + t * CB
            pltpu.sync_copy(src_hbm.at[pl.ds(base, CB)], src_v)
            pltpu.sync_copy(dst_hbm.at[pl.ds(base, CB)], dst_v)
            pltpu.sync_copy(et_hbm.at[pl.ds(base, CB)], et_v)
            pltpu.sync_copy(ex_hbm.at[pl.ds(base, CB), :], ex_v)
            for j in range(CB // 16):
                sl = pl.ds(j * 16, 16)
                idx_v[sl] = (et_v[sl] * NP + src_v[sl]) * 2 + hh
            cp0 = pltpu.async_copy(h_hbm.at[idx_v], rows_v, sem0)
            cp1 = pltpu.async_copy(dena_hbm.at[dst_v], da_v, sem1)
            cp2 = pltpu.async_copy(denb_hbm.at[dst_v], db_v, sem2)
            cp0.wait()
            cp1.wait()
            cp2.wait()
            h0 = 2 * hh
            for r in range(CB):
                al = ex_v[r, :] / (da_v[r, :] + db_v[r, :])
                s0 = jnp.broadcast_to(al[h0:h0 + 1], (16,))
                s1 = jnp.broadcast_to(al[h0 + 1:h0 + 2], (16,))
                for v in range(4):
                    sl = pl.ds(v * 16, 16)
                    rows_v[r, sl] = rows_v[r, sl] * s0
                for v in range(4):
                    sl = pl.ds(64 + v * 16, 16)
                    rows_v[r, sl] = rows_v[r, sl] * s1
            pltpu.sync_copy(rows_v, acc_sp.at[dst_v], add=True)
            return carry

        @pl.when(cid == 0)
        def _():
            lax.fori_loop(0, EPW_B // CB,
                          functools.partial(chunk, hh=0), 0)

        @pl.when(cid == 1)
        def _():
            lax.fori_loop(0, EPW_B // CB,
                          functools.partial(chunk, hh=1), 0)

        plsc.subcore_barrier()
        pltpu.sync_copy(acc_sp.at[pl.ds(sid * STRIPE, STRIPE), :],
                        out_hbm.at[cid, pl.ds(sid * STRIPE, STRIPE), :])

    return k(h_flat, ex, den_a, den_b, src, dst, et, zeros_d)


def _edge_phase(h_all, aq_node, ak_node, src, dst, et, zeros_h, zeros_d):
    aqf = jnp.pad(aq_node.reshape(R * NP, H), ((0, 0), (0, HP - H)))
    akf = jnp.pad(ak_node.reshape(R * NP, H), ((0, 0), (0, HP - H)))
    ex, den = _edge_attention_sc(aqf, akf, src, dst, et, zeros_h)
    h_flat = h_all.reshape(R * NP * 2, 128)
    out = _edge_apply_sc(h_flat, ex, den[0], den[1], src, dst, et, zeros_d)
    return out


def kernel(x, edge_index, edge_type, batch, W1, q1, k1, b1, W2, q2, k2, b2,
           lin1_w, lin1_b, lin2_w, lin2_b):
    src = edge_index[0]
    dst = edge_index[1]
    et = edge_type

    x_pad = jnp.pad(x, ((0, NP - N), (0, 0)))
    b1_2d = b1.reshape(1, HD)
    b2_2d = b2.reshape(1, D)
    zeros_h = jnp.zeros((NP, HP), jnp.float32)
    zeros_d = jnp.zeros((NP, 128), jnp.float32)

    h1, aq1, ak1 = _relational_transform(x_pad, W1, q1, k1, b1_2d, act=False)
    out1 = _edge_phase(h1, aq1, ak1, src, dst, et, zeros_h, zeros_d)

    h2, aq2, ak2 = _relational_transform2(out1[0], out1[1], W2, q2, k2, b1_2d)
    out2 = _edge_phase(h2, aq2, ak2, src, dst, et, zeros_h, zeros_d)

    batch_pad = jnp.pad(batch, (0, NP - N), constant_values=G)
    batch3 = batch_pad.reshape(NP // BN, 1, BN)
    batch_col = batch_pad.reshape(NP // BN, BN, 1)
    rows = batch_pad.reshape(NP // BN, BN)
    grange = jnp.stack(
        [rows[:, 0], jnp.minimum(rows[:, -1], G - 1)], axis=1
    ).astype(jnp.int32)
    out = pl.pallas_call(
        _pool_body,
        grid=(NP // BN,),
        in_specs=[
            pl.BlockSpec((BN, 128), lambda i: (i, 0)),
            pl.BlockSpec((BN, 128), lambda i: (i, 0)),
            pl.BlockSpec((1, D), lambda i: (0, 0)),
            pl.BlockSpec((1, 1, BN), lambda i: (i, 0, 0)),
            pl.BlockSpec((1, BN, 1), lambda i: (i, 0, 0)),
            pl.BlockSpec(memory_space=pltpu.SMEM),
            pl.BlockSpec((D, D), lambda i: (0, 0)),
            pl.BlockSpec((1, D), lambda i: (0, 0)),
            pl.BlockSpec((D, G), lambda i: (0, 0)),
            pl.BlockSpec((1, G), lambda i: (0, 0)),
        ],
        out_specs=pl.BlockSpec((G, G), lambda i: (0, 0)),
        out_shape=jax.ShapeDtypeStruct((G, G), jnp.float32),
        scratch_shapes=[
            pltpu.VMEM((G, D), jnp.float32),
            pltpu.VMEM((G, D), jnp.float32),
            pltpu.VMEM((G, D), jnp.float32),
        ],
    )(out2[0], out2[1], b2_2d, batch3, batch_col, grange, lin1_w,
      lin1_b.reshape(1, D), lin2_w, lin2_b.reshape(1, G))
    return out


# speedup vs baseline: 36.3930x; 1.5762x over previous
import functools

import jax
import jax.numpy as jnp
from jax import lax
from jax.experimental import pallas as pl
from jax.experimental.pallas import tpu as pltpu
from jax.experimental.pallas import tpu_sc as plsc

N = 10000
E = 320000
R = 8
H = 4
D = 64
G = 64
HD = H * D

NP = 10240
BM = 512
BN = 256


def _mm_body(x_ref, w_ref, q_ref, k_ref, b_ref, h_ref, aq_ref, ak_ref, *, act):
    xb = x_ref[...]
    if act:
        xb = xb + b_ref[...]
        xb = jnp.where(xb > 0, xb, jnp.exp(jnp.minimum(xb, 0.0)) - 1.0)
    h = jnp.dot(xb, w_ref[0], preferred_element_type=jnp.float32)
    h4 = h.reshape(BM, H, D)
    aq = jnp.sum(h4 * q_ref[0][None], axis=-1)
    ak = jnp.sum(h4 * k_ref[0][None], axis=-1)
    h_ref[0] = h
    aq_ref[0] = aq
    ak_ref[0] = ak


def _relational_transform(x_pad, W, q_att, k_att, bias2d, act):
    F = x_pad.shape[1]
    grid = (R, NP // BM)
    out_shapes = (
        jax.ShapeDtypeStruct((R, NP, HD), jnp.float32),
        jax.ShapeDtypeStruct((R, NP, H), jnp.float32),
        jax.ShapeDtypeStruct((R, NP, H), jnp.float32),
    )
    return pl.pallas_call(
        functools.partial(_mm_body, act=act),
        grid=grid,
        in_specs=[
            pl.BlockSpec((BM, F), lambda r, i: (i, 0)),
            pl.BlockSpec((1, F, HD), lambda r, i: (r, 0, 0)),
            pl.BlockSpec((1, H, D), lambda r, i: (r, 0, 0)),
            pl.BlockSpec((1, H, D), lambda r, i: (r, 0, 0)),
            pl.BlockSpec((1, HD), lambda r, i: (0, 0)),
        ],
        out_specs=(
            pl.BlockSpec((1, BM, HD), lambda r, i: (r, i, 0)),
            pl.BlockSpec((1, BM, H), lambda r, i: (r, i, 0)),
            pl.BlockSpec((1, BM, H), lambda r, i: (r, i, 0)),
        ),
        out_shape=out_shapes,
    )(x_pad, W, q_att, k_att, bias2d)


def _mm_body2(a_ref, b_ref, w_ref, q_ref, k_ref, bias_ref,
              h_ref, aq_ref, ak_ref):
    bias = bias_ref[...]
    xa = a_ref[...] + bias[:, 0:128]
    xa = jnp.where(xa > 0, xa, jnp.exp(jnp.minimum(xa, 0.0)) - 1.0)
    xb = b_ref[...] + bias[:, 128:256]
    xb = jnp.where(xb > 0, xb, jnp.exp(jnp.minimum(xb, 0.0)) - 1.0)
    w = w_ref[0]
    h = (jnp.dot(xa, w[0:128], preferred_element_type=jnp.float32)
         + jnp.dot(xb, w[128:256], preferred_element_type=jnp.float32))
    h4 = h.reshape(BM, H, D)
    aq_ref[0] = jnp.sum(h4 * q_ref[0][None], axis=-1)
    ak_ref[0] = jnp.sum(h4 * k_ref[0][None], axis=-1)
    h_ref[0] = h


def _relational_transform2(a, b, W, q_att, k_att, bias2d):
    grid = (R, NP // BM)
    out_shapes = (
        jax.ShapeDtypeStruct((R, NP, HD), jnp.float32),
        jax.ShapeDtypeStruct((R, NP, H), jnp.float32),
        jax.ShapeDtypeStruct((R, NP, H), jnp.float32),
    )
    return pl.pallas_call(
        _mm_body2,
        grid=grid,
        in_specs=[
            pl.BlockSpec((BM, 128), lambda r, i: (i, 0)),
            pl.BlockSpec((BM, 128), lambda r, i: (i, 0)),
            pl.BlockSpec((1, HD, HD), lambda r, i: (r, 0, 0)),
            pl.BlockSpec((1, H, D), lambda r, i: (r, 0, 0)),
            pl.BlockSpec((1, H, D), lambda r, i: (r, 0, 0)),
            pl.BlockSpec((1, HD), lambda r, i: (0, 0)),
        ],
        out_specs=(
            pl.BlockSpec((1, BM, HD), lambda r, i: (r, i, 0)),
            pl.BlockSpec((1, BM, H), lambda r, i: (r, i, 0)),
            pl.BlockSpec((1, BM, H), lambda r, i: (r, i, 0)),
        ),
        out_shape=out_shapes,
    )(a, b, W, q_att, k_att, bias2d)


def _pool_body(pa_ref, pb_ref, b2_ref, batch_ref, bcol_ref, gr_ref, l1w_ref,
               l1b_ref, l2w_ref, l2b_ref, out_ref, acc_s, acc_c, acc_m):
    i = pl.program_id(0)
    nb = pl.num_programs(0)

    @pl.when(i == 0)
    def _():
        acc_s[...] = jnp.zeros((G, D), jnp.float32)
        acc_c[...] = jnp.zeros((G, D), jnp.float32)
        acc_m[...] = jnp.full((G, D), -jnp.inf, jnp.float32)

    pa = pa_ref[...]
    pb = pb_ref[...]
    zb = (pa[:, 0:64] + pa[:, 64:128] + pb[:, 0:64] + pb[:, 64:128]) * 0.25
    zb = zb + b2_ref[...]
    xb = jnp.where(zb > 0, zb, jnp.exp(jnp.minimum(zb, 0.0)) - 1.0)
    bb = batch_ref[0]
    gids = lax.broadcasted_iota(jnp.int32, (G, BN), 0)
    mask = bb == gids
    maskf = mask.astype(jnp.float32)
    acc_s[...] += jnp.dot(maskf, xb, preferred_element_type=jnp.float32)
    acc_c[...] += jnp.dot(maskf, jnp.ones((BN, D), jnp.float32),
                          preferred_element_type=jnp.float32)
    bcol = bcol_ref[0]
    g_lo = gr_ref[i, 0]
    g_hi = gr_ref[i, 1]

    def _max_body(g, carry):
        m = jnp.where(bcol == g, xb, -jnp.inf)
        mx_g = jnp.max(m, axis=0, keepdims=True)
        acc_m[pl.ds(g, 1), :] = jnp.maximum(acc_m[pl.ds(g, 1), :], mx_g)
        return carry

    lax.fori_loop(g_lo, g_hi + 1, _max_body, 0)

    @pl.when(i == nb - 1)
    def _():
        mean = acc_s[...] / jnp.maximum(acc_c[...], 1.0)
        mxv = acc_m[...]
        mxv = jnp.where(mxv == -jnp.inf, 0.0, mxv)
        pool = mean + mxv
        h = jnp.maximum(
            jnp.dot(pool, l1w_ref[...], preferred_element_type=jnp.float32)
            + l1b_ref[...], 0.0)
        out_ref[...] = (
            jnp.dot(h, l2w_ref[...], preferred_element_type=jnp.float32)
            + l2b_ref[...])


CB = 80
NTILES = 32
EPW_A = E // NTILES
EPW_B = E // 16
STRIPE = NP // 16

_MESH = plsc.VectorSubcoreMesh(core_axis_name="c", subcore_axis_name="s")
_SC_PARAMS = pltpu.CompilerParams(use_tc_tiling_on_sc=False)


def _lanes():
    return lax.iota(jnp.int32, 16)


HP = 16


def _edge_attention_sc(aqf, akf, src, dst, et, zeros_h):
    @functools.partial(
        pl.kernel,
        mesh=_MESH,
        compiler_params=_SC_PARAMS,
        out_type=(
            jax.ShapeDtypeStruct((E, HP), jnp.float32),
            jax.ShapeDtypeStruct((2, NP, HP), jnp.float32),
        ),
        scratch_types=[
            pltpu.VMEM((EPW_A,), jnp.int32),
            pltpu.VMEM((EPW_A,), jnp.int32),
            pltpu.VMEM((EPW_A,), jnp.int32),
            pltpu.VMEM((2, CB), jnp.int32),
            pltpu.VMEM((2, CB), jnp.int32),
            pltpu.VMEM((2, CB), jnp.int32),
            pltpu.VMEM((2, CB, HP), jnp.float32),
            pltpu.VMEM((2, CB, HP), jnp.float32),
            pltpu.VMEM((2, CB, HP), jnp.float32),
            pltpu.VMEM_SHARED((NP, HP), jnp.float32),
            pltpu.SemaphoreType.DMA,
            pltpu.SemaphoreType.DMA,
            pltpu.SemaphoreType.DMA,
            pltpu.SemaphoreType.DMA,
        ],
    )
    def k(aq_hbm, ak_hbm, src_hbm, dst_hbm, et_hbm, z_hbm,
          ex_hbm, den_hbm,
          srcall_v, dstall_v, etall_v, dstc_v, idxq_v, idxk_v,
          aq_v, ak_v, ex_v, den_sp, semq0, semq1, semk0, semk1):
        semq = (semq0, semq1)
        semk = (semk0, semk1)
        cid = lax.axis_index("c")
        sid = lax.axis_index("s")
        wid = sid * 2 + cid

        pltpu.sync_copy(z_hbm.at[pl.ds(sid * STRIPE, STRIPE), :],
                        den_sp.at[pl.ds(sid * STRIPE, STRIPE), :])
        plsc.subcore_barrier()

        NCH = EPW_A // CB
        tbase = wid * EPW_A
        pltpu.sync_copy(src_hbm.at[pl.ds(tbase, EPW_A)], srcall_v)
        pltpu.sync_copy(dst_hbm.at[pl.ds(tbase, EPW_A)], dstall_v)
        pltpu.sync_copy(et_hbm.at[pl.ds(tbase, EPW_A)], etall_v)

        def issue(c, b):
            for j in range(CB // 16):
                sl = pl.ds(c * CB + j * 16, 16)
                so = pl.ds(j * 16, 16)
                etj = etall_v[sl] * NP
                dstj = dstall_v[sl]
                dstc_v[b, so] = dstj
                idxq_v[b, so] = etj + dstj
                idxk_v[b, so] = etj + srcall_v[sl]
            pltpu.async_copy(aq_hbm.at[idxq_v.at[b]], aq_v.at[b], semq[b])
            pltpu.async_copy(ak_hbm.at[idxk_v.at[b]], ak_v.at[b], semk[b])

        issue(0, 0)
        issue(1, 1)

        def chunk(c, b):
            pltpu.make_async_copy(aq_hbm.at[idxq_v.at[b]], aq_v.at[b],
                                  semq[b]).wait()
            pltpu.make_async_copy(ak_hbm.at[idxk_v.at[b]], ak_v.at[b],
                                  semk[b]).wait()
            for r in range(CB):
                ev = aq_v[b, r, :] + ak_v[b, r, :]
                ev = jnp.where(ev >= 0, ev, 0.2 * ev)
                ex_v[b, r, :] = jnp.exp(ev)
            pltpu.sync_copy(ex_v.at[b],
                            ex_hbm.at[pl.ds(tbase + c * CB, CB), :])
            pltpu.sync_copy(ex_v.at[b], den_sp.at[dstc_v.at[b]], add=True)

            @pl.when(c + 2 < NCH)
            def _():
                issue(c + 2, b)

        def pair(t, carry):
            chunk(2 * t, 0)
            chunk(2 * t + 1, 1)
            return carry

        lax.fori_loop(0, NCH // 2, pair, 0)
        if NCH % 2:
            chunk(NCH - 1, 0)
        plsc.subcore_barrier()
        pltpu.sync_copy(den_sp.at[pl.ds(sid * STRIPE, STRIPE), :],
                        den_hbm.at[cid, pl.ds(sid * STRIPE, STRIPE), :])

    return k(aqf, akf, src, dst, et, zeros_h)


def _edge_apply_sc(h_flat, ex, rden, src, dst, et, zeros_d):
    @functools.partial(
        pl.kernel,
        mesh=_MESH,
        compiler_params=_SC_PARAMS,
        out_type=jax.ShapeDtypeStruct((2, NP, 128), jnp.float32),
        scratch_types=[
            pltpu.VMEM((2, CB), jnp.int32),
            pltpu.VMEM((2, CB), jnp.int32),
            pltpu.VMEM((2, CB), jnp.int32),
            pltpu.VMEM((2, CB), jnp.int32),
            pltpu.VMEM((2, CB), jnp.int32),
            pltpu.VMEM((2, CB, HP), jnp.float32),
            pltpu.VMEM((2, CB, HP), jnp.float32),
            pltpu.VMEM((2, CB, 128), jnp.float32),
            pltpu.VMEM_SHARED((NP, 128), jnp.float32),
            pltpu.SemaphoreType.DMA,
            pltpu.SemaphoreType.DMA,
            pltpu.SemaphoreType.DMA,
            pltpu.SemaphoreType.DMA,
            pltpu.SemaphoreType.DMA,
            pltpu.SemaphoreType.DMA,
            pltpu.SemaphoreType.DMA,
            pltpu.SemaphoreType.DMA,
        ],
    )
    def k(h_hbm, ex_hbm, rden_hbm, src_hbm, dst_hbm, et_hbm, z_hbm,
          out_hbm,
          src_v, dst_v, et_v, dstc_v, idx_v, ex_v, rd_v, rows_v,
          acc_sp, seml0, seml1, semr0, semr1, semd0, semd1, seme0, seme1):
        seml = (seml0, seml1)
        semr = (semr0, semr1)
        semd = (semd0, semd1)
        seme = (seme0, seme1)
        cid = lax.axis_index("c")
        sid = lax.axis_index("s")

        pltpu.sync_copy(z_hbm.at[pl.ds(sid * STRIPE, STRIPE), :],
                        acc_sp.at[pl.ds(sid * STRIPE, STRIPE), :])
        plsc.subcore_barrier()

        NCH = EPW_B // CB
        tbase = sid * EPW_B

        def issue_linear(c, b):
            base = pl.ds(tbase + c * CB, CB)
            pltpu.async_copy(src_hbm.at[base], src_v.at[b], seml[b])
            pltpu.async_copy(dst_hbm.at[base], dst_v.at[b], seml[b])
            pltpu.async_copy(et_hbm.at[base], et_v.at[b], seml[b])

        def wait_linear(c, b):
            base = pl.ds(tbase + c * CB, CB)
            pltpu.make_async_copy(src_hbm.at[base], src_v.at[b],
                                  seml[b]).wait()
            pltpu.make_async_copy(dst_hbm.at[base], dst_v.at[b],
                                  seml[b]).wait()
            pltpu.make_async_copy(et_hbm.at[base], et_v.at[b],
                                  seml[b]).wait()

        def issue_gather(c, b, hh):
            for j in range(CB // 16):
                sl = pl.ds(j * 16, 16)
                dstc_v[b, sl] = dst_v[b, sl]
                idx_v[b, sl] = (et_v[b, sl] * NP + src_v[b, sl]) * 2 + hh
            pltpu.async_copy(h_hbm.at[idx_v.at[b]], rows_v.at[b], semr[b])
            pltpu.async_copy(rden_hbm.at[dstc_v.at[b]], rd_v.at[b], semd[b])
            pltpu.async_copy(ex_hbm.at[pl.ds(tbase + c * CB, CB), :],
                             ex_v.at[b], seme[b])

        def chunk(c, b, hh):
            @pl.when(c + 1 < NCH)
            def _():
                wait_linear(c + 1, 1 - b)
                issue_gather(c + 1, 1 - b, hh)

            pltpu.make_async_copy(h_hbm.at[idx_v.at[b]], rows_v.at[b],
                                  semr[b]).wait()
            pltpu.make_async_copy(rden_hbm.at[dstc_v.at[b]], rd_v.at[b],
                                  semd[b]).wait()
            pltpu.make_async_copy(ex_hbm.at[pl.ds(tbase + c * CB, CB), :],
                                  ex_v.at[b], seme[b]).wait()
            h0 = 2 * hh
            for r in range(CB):
                al = ex_v[b, r, :] * rd_v[b, r, :]
                s0 = jnp.broadcast_to(al[h0:h0 + 1], (16,))
                s1 = jnp.broadcast_to(al[h0 + 1:h0 + 2], (16,))
                for v in range(4):
                    sl = pl.ds(v * 16, 16)
                    rows_v[b, r, sl] = rows_v[b, r, sl] * s0
                for v in range(4):
                    sl = pl.ds(64 + v * 16, 16)
                    rows_v[b, r, sl] = rows_v[b, r, sl] * s1
            pltpu.sync_copy(rows_v.at[b], acc_sp.at[dstc_v.at[b]], add=True)

            @pl.when(c + 2 < NCH)
            def _():
                issue_linear(c + 2, b)

        def run(hh):
            issue_linear(0, 0)
            wait_linear(0, 0)
            issue_gather(0, 0, hh)
            issue_linear(1, 1)

            def pair(t, carry):
                chunk(2 * t, 0, hh)
                chunk(2 * t + 1, 1, hh)
                return carry

            lax.fori_loop(0, NCH // 2, pair, 0)
            if NCH % 2:
                chunk(NCH - 1, 0, hh)

        @pl.when(cid == 0)
        def _():
            run(0)

        @pl.when(cid == 1)
        def _():
            run(1)

        plsc.subcore_barrier()
        pltpu.sync_copy(acc_sp.at[pl.ds(sid * STRIPE, STRIPE), :],
                        out_hbm.at[cid, pl.ds(sid * STRIPE, STRIPE), :])

    return k(h_flat, ex, rden, src, dst, et, zeros_d)


def _rden_body(den_ref, out_ref):
    s = den_ref[0] + den_ref[1]
    out_ref[...] = 1.0 / jnp.where(s == 0.0, 1.0, s)


def _den_recip(den):
    BR = 2048
    return pl.pallas_call(
        _rden_body,
        grid=(NP // BR,),
        in_specs=[pl.BlockSpec((2, BR, HP), lambda i: (0, i, 0))],
        out_specs=pl.BlockSpec((BR, HP), lambda i: (i, 0)),
        out_shape=jax.ShapeDtypeStruct((NP, HP), jnp.float32),
    )(den)


def _edge_phase(h_all, aq_node, ak_node, src, dst, et, zeros_h, zeros_d):
    aqf = jnp.pad(aq_node.reshape(R * NP, H), ((0, 0), (0, HP - H)))
    akf = jnp.pad(ak_node.reshape(R * NP, H), ((0, 0), (0, HP - H)))
    ex, den = _edge_attention_sc(aqf, akf, src, dst, et, zeros_h)
    rden = _den_recip(den)
    h_flat = h_all.reshape(R * NP * 2, 128)
    out = _edge_apply_sc(h_flat, ex, rden, src, dst, et, zeros_d)
    return out


def kernel(x, edge_index, edge_type, batch, W1, q1, k1, b1, W2, q2, k2, b2,
           lin1_w, lin1_b, lin2_w, lin2_b):
    src = edge_index[0]
    dst = edge_index[1]
    et = edge_type

    x_pad = jnp.pad(x, ((0, NP - N), (0, 0)))
    b1_2d = b1.reshape(1, HD)
    b2_2d = b2.reshape(1, D)
    zeros_h = jnp.zeros((NP, HP), jnp.float32)
    zeros_d = jnp.zeros((NP, 128), jnp.float32)

    h1, aq1, ak1 = _relational_transform(x_pad, W1, q1, k1, b1_2d, act=False)
    out1 = _edge_phase(h1, aq1, ak1, src, dst, et, zeros_h, zeros_d)

    h2, aq2, ak2 = _relational_transform2(out1[0], out1[1], W2, q2, k2, b1_2d)
    out2 = _edge_phase(h2, aq2, ak2, src, dst, et, zeros_h, zeros_d)

    batch_pad = jnp.pad(batch, (0, NP - N), constant_values=G)
    batch3 = batch_pad.reshape(NP // BN, 1, BN)
    batch_col = batch_pad.reshape(NP // BN, BN, 1)
    rows = batch_pad.reshape(NP // BN, BN)
    grange = jnp.stack(
        [rows[:, 0], jnp.minimum(rows[:, -1], G - 1)], axis=1
    ).astype(jnp.int32)
    out = pl.pallas_call(
        _pool_body,
        grid=(NP // BN,),
        in_specs=[
            pl.BlockSpec((BN, 128), lambda i: (i, 0)),
            pl.BlockSpec((BN, 128), lambda i: (i, 0)),
            pl.BlockSpec((1, D), lambda i: (0, 0)),
            pl.BlockSpec((1, 1, BN), lambda i: (i, 0, 0)),
            pl.BlockSpec((1, BN, 1), lambda i: (i, 0, 0)),
            pl.BlockSpec(memory_space=pltpu.SMEM),
            pl.BlockSpec((D, D), lambda i: (0, 0)),
            pl.BlockSpec((1, D), lambda i: (0, 0)),
            pl.BlockSpec((D, G), lambda i: (0, 0)),
            pl.BlockSpec((1, G), lambda i: (0, 0)),
        ],
        out_specs=pl.BlockSpec((G, G), lambda i: (0, 0)),
        out_shape=jax.ShapeDtypeStruct((G, G), jnp.float32),
        scratch_shapes=[
            pltpu.VMEM((G, D), jnp.float32),
            pltpu.VMEM((G, D), jnp.float32),
            pltpu.VMEM((G, D), jnp.float32),
        ],
    )(out2[0], out2[1], b2_2d, batch3, batch_col, grange, lin1_w,
      lin1_b.reshape(1, D), lin2_w, lin2_b.reshape(1, G))
    return out


# speedup vs baseline: 40.4778x; 1.1122x over previous
import functools

import jax
import jax.numpy as jnp
from jax import lax
from jax.experimental import pallas as pl
from jax.experimental.pallas import tpu as pltpu
from jax.experimental.pallas import tpu_sc as plsc

N = 10000
E = 320000
R = 8
H = 4
D = 64
G = 64
HD = H * D

NP = 10240
BM = 512
BN = 256
HP = 16


def _mm_body(x_ref, w_ref, q_ref, k_ref, b_ref, h_ref, aq_ref, ak_ref, *, act):
    xb = x_ref[...]
    if act:
        xb = xb + b_ref[...]
        xb = jnp.where(xb > 0, xb, jnp.exp(jnp.minimum(xb, 0.0)) - 1.0)
    h = jnp.dot(xb, w_ref[0], preferred_element_type=jnp.float32)
    h4 = h.reshape(BM, H, D)
    aq = jnp.sum(h4 * q_ref[0][None], axis=-1)
    ak = jnp.sum(h4 * k_ref[0][None], axis=-1)
    zpad = jnp.zeros((BM, HP - H), jnp.float32)
    h_ref[0] = h
    aq_ref[0] = jnp.concatenate([aq, zpad], axis=1)
    ak_ref[0] = jnp.concatenate([ak, zpad], axis=1)


def _relational_transform(x_pad, W, q_att, k_att, bias2d, act):
    F = x_pad.shape[1]
    grid = (R, NP // BM)
    out_shapes = (
        jax.ShapeDtypeStruct((R, NP, HD), jnp.float32),
        jax.ShapeDtypeStruct((R, NP, HP), jnp.float32),
        jax.ShapeDtypeStruct((R, NP, HP), jnp.float32),
    )
    return pl.pallas_call(
        functools.partial(_mm_body, act=act),
        grid=grid,
        in_specs=[
            pl.BlockSpec((BM, F), lambda r, i: (i, 0)),
            pl.BlockSpec((1, F, HD), lambda r, i: (r, 0, 0)),
            pl.BlockSpec((1, H, D), lambda r, i: (r, 0, 0)),
            pl.BlockSpec((1, H, D), lambda r, i: (r, 0, 0)),
            pl.BlockSpec((1, HD), lambda r, i: (0, 0)),
        ],
        out_specs=(
            pl.BlockSpec((1, BM, HD), lambda r, i: (r, i, 0)),
            pl.BlockSpec((1, BM, HP), lambda r, i: (r, i, 0)),
            pl.BlockSpec((1, BM, HP), lambda r, i: (r, i, 0)),
        ),
        out_shape=out_shapes,
    )(x_pad, W, q_att, k_att, bias2d)


def _mm_body2(a_ref, b_ref, w_ref, q_ref, k_ref, bias_ref,
              h_ref, aq_ref, ak_ref):
    bias = bias_ref[...]
    xa = a_ref[...] + bias[:, 0:128]
    xa = jnp.where(xa > 0, xa, jnp.exp(jnp.minimum(xa, 0.0)) - 1.0)
    xb = b_ref[...] + bias[:, 128:256]
    xb = jnp.where(xb > 0, xb, jnp.exp(jnp.minimum(xb, 0.0)) - 1.0)
    w = w_ref[0]
    h = (jnp.dot(xa, w[0:128], preferred_element_type=jnp.float32)
         + jnp.dot(xb, w[128:256], preferred_element_type=jnp.float32))
    h4 = h.reshape(BM, H, D)
    zpad = jnp.zeros((BM, HP - H), jnp.float32)
    aq_ref[0] = jnp.concatenate(
        [jnp.sum(h4 * q_ref[0][None], axis=-1), zpad], axis=1)
    ak_ref[0] = jnp.concatenate(
        [jnp.sum(h4 * k_ref[0][None], axis=-1), zpad], axis=1)
    h_ref[0] = h


def _relational_transform2(a, b, W, q_att, k_att, bias2d):
    grid = (R, NP // BM)
    out_shapes = (
        jax.ShapeDtypeStruct((R, NP, HD), jnp.float32),
        jax.ShapeDtypeStruct((R, NP, HP), jnp.float32),
        jax.ShapeDtypeStruct((R, NP, HP), jnp.float32),
    )
    return pl.pallas_call(
        _mm_body2,
        grid=grid,
        in_specs=[
            pl.BlockSpec((BM, 128), lambda r, i: (i, 0)),
            pl.BlockSpec((BM, 128), lambda r, i: (i, 0)),
            pl.BlockSpec((1, HD, HD), lambda r, i: (r, 0, 0)),
            pl.BlockSpec((1, H, D), lambda r, i: (r, 0, 0)),
            pl.BlockSpec((1, H, D), lambda r, i: (r, 0, 0)),
            pl.BlockSpec((1, HD), lambda r, i: (0, 0)),
        ],
        out_specs=(
            pl.BlockSpec((1, BM, HD), lambda r, i: (r, i, 0)),
            pl.BlockSpec((1, BM, HP), lambda r, i: (r, i, 0)),
            pl.BlockSpec((1, BM, HP), lambda r, i: (r, i, 0)),
        ),
        out_shape=out_shapes,
    )(a, b, W, q_att, k_att, bias2d)


def _pool_body(pa_ref, pb_ref, b2_ref, batch_ref, bcol_ref, gr_ref, l1w_ref,
               l1b_ref, l2w_ref, l2b_ref, out_ref, acc_s, acc_c, acc_m):
    i = pl.program_id(0)
    nb = pl.num_programs(0)

    @pl.when(i == 0)
    def _():
        acc_s[...] = jnp.zeros((G, D), jnp.float32)
        acc_c[...] = jnp.zeros((G, D), jnp.float32)
        acc_m[...] = jnp.full((G, D), -jnp.inf, jnp.float32)

    pa = pa_ref[...]
    pb = pb_ref[...]
    zb = (pa[:, 0:64] + pa[:, 64:128] + pb[:, 0:64] + pb[:, 64:128]) * 0.25
    zb = zb + b2_ref[...]
    xb = jnp.where(zb > 0, zb, jnp.exp(jnp.minimum(zb, 0.0)) - 1.0)
    bb = batch_ref[0]
    gids = lax.broadcasted_iota(jnp.int32, (G, BN), 0)
    mask = bb == gids
    maskf = mask.astype(jnp.float32)
    acc_s[...] += jnp.dot(maskf, xb, preferred_element_type=jnp.float32)
    acc_c[...] += jnp.dot(maskf, jnp.ones((BN, D), jnp.float32),
                          preferred_element_type=jnp.float32)
    bcol = bcol_ref[0]
    g_lo = gr_ref[i, 0]
    g_hi = gr_ref[i, 1]

    def _max_body(g, carry):
        m = jnp.where(bcol == g, xb, -jnp.inf)
        mx_g = jnp.max(m, axis=0, keepdims=True)
        acc_m[pl.ds(g, 1), :] = jnp.maximum(acc_m[pl.ds(g, 1), :], mx_g)
        return carry

    lax.fori_loop(g_lo, g_hi + 1, _max_body, 0)

    @pl.when(i == nb - 1)
    def _():
        mean = acc_s[...] / jnp.maximum(acc_c[...], 1.0)
        mxv = acc_m[...]
        mxv = jnp.where(mxv == -jnp.inf, 0.0, mxv)
        pool = mean + mxv
        h = jnp.maximum(
            jnp.dot(pool, l1w_ref[...], preferred_element_type=jnp.float32)
            + l1b_ref[...], 0.0)
        out_ref[...] = (
            jnp.dot(h, l2w_ref[...], preferred_element_type=jnp.float32)
            + l2b_ref[...])


CB = 80
NTILES = 32
EPW_A = E // NTILES
EPW_B = E // 16
STRIPE = NP // 16

_MESH = plsc.VectorSubcoreMesh(core_axis_name="c", subcore_axis_name="s")
_SC_PARAMS = pltpu.CompilerParams(use_tc_tiling_on_sc=False)


def _lanes():
    return lax.iota(jnp.int32, 16)


def _edge_attention_sc(aqf, akf, src, dst, et, zeros_h):
    @functools.partial(
        pl.kernel,
        mesh=_MESH,
        compiler_params=_SC_PARAMS,
        out_type=(
            jax.ShapeDtypeStruct((E, HP), jnp.float32),
            jax.ShapeDtypeStruct((2, NP, HP), jnp.float32),
        ),
        scratch_types=[
            pltpu.VMEM((EPW_A,), jnp.int32),
            pltpu.VMEM((EPW_A,), jnp.int32),
            pltpu.VMEM((EPW_A,), jnp.int32),
            pltpu.VMEM((2, CB), jnp.int32),
            pltpu.VMEM((2, CB), jnp.int32),
            pltpu.VMEM((2, CB), jnp.int32),
            pltpu.VMEM((2, CB, HP), jnp.float32),
            pltpu.VMEM((2, CB, HP), jnp.float32),
            pltpu.VMEM((2, CB, HP), jnp.float32),
            pltpu.VMEM_SHARED((NP, HP), jnp.float32),
            pltpu.SemaphoreType.DMA,
            pltpu.SemaphoreType.DMA,
            pltpu.SemaphoreType.DMA,
            pltpu.SemaphoreType.DMA,
        ],
    )
    def k(aq_hbm, ak_hbm, src_hbm, dst_hbm, et_hbm, z_hbm,
          ex_hbm, den_hbm,
          srcall_v, dstall_v, etall_v, dstc_v, idxq_v, idxk_v,
          aq_v, ak_v, ex_v, den_sp, semq0, semq1, semk0, semk1):
        semq = (semq0, semq1)
        semk = (semk0, semk1)
        cid = lax.axis_index("c")
        sid = lax.axis_index("s")
        wid = sid * 2 + cid

        pltpu.sync_copy(z_hbm.at[pl.ds(sid * STRIPE, STRIPE), :],
                        den_sp.at[pl.ds(sid * STRIPE, STRIPE), :])
        plsc.subcore_barrier()

        NCH = EPW_A // CB
        tbase = wid * EPW_A
        pltpu.sync_copy(src_hbm.at[pl.ds(tbase, EPW_A)], srcall_v)
        pltpu.sync_copy(dst_hbm.at[pl.ds(tbase, EPW_A)], dstall_v)
        pltpu.sync_copy(et_hbm.at[pl.ds(tbase, EPW_A)], etall_v)

        def issue(c, b):
            for j in range(CB // 16):
                sl = pl.ds(c * CB + j * 16, 16)
                so = pl.ds(j * 16, 16)
                etj = etall_v[sl] * NP
                dstj = dstall_v[sl]
                dstc_v[b, so] = dstj
                idxq_v[b, so] = etj + dstj
                idxk_v[b, so] = etj + srcall_v[sl]
            pltpu.async_copy(aq_hbm.at[idxq_v.at[b]], aq_v.at[b], semq[b])
            pltpu.async_copy(ak_hbm.at[idxk_v.at[b]], ak_v.at[b], semk[b])

        issue(0, 0)
        issue(1, 1)

        def chunk(c, b):
            pltpu.make_async_copy(aq_hbm.at[idxq_v.at[b]], aq_v.at[b],
                                  semq[b]).wait()
            pltpu.make_async_copy(ak_hbm.at[idxk_v.at[b]], ak_v.at[b],
                                  semk[b]).wait()
            for r in range(CB):
                ev = aq_v[b, r, :] + ak_v[b, r, :]
                ev = jnp.where(ev >= 0, ev, 0.2 * ev)
                ex_v[b, r, :] = jnp.exp(ev)
            pltpu.sync_copy(ex_v.at[b],
                            ex_hbm.at[pl.ds(tbase + c * CB, CB), :])
            pltpu.sync_copy(ex_v.at[b], den_sp.at[dstc_v.at[b]], add=True)

            @pl.when(c + 2 < NCH)
            def _():
                issue(c + 2, b)

        def pair(t, carry):
            chunk(2 * t, 0)
            chunk(2 * t + 1, 1)
            return carry

        lax.fori_loop(0, NCH // 2, pair, 0)
        if NCH % 2:
            chunk(NCH - 1, 0)
        plsc.subcore_barrier()
        pltpu.sync_copy(den_sp.at[pl.ds(sid * STRIPE, STRIPE), :],
                        den_hbm.at[cid, pl.ds(sid * STRIPE, STRIPE), :])

    return k(aqf, akf, src, dst, et, zeros_h)


def _edge_apply_sc(h_flat, ex, rden, src, dst, et, zeros_d):
    @functools.partial(
        pl.kernel,
        mesh=_MESH,
        compiler_params=_SC_PARAMS,
        out_type=jax.ShapeDtypeStruct((2, NP, 128), jnp.float32),
        scratch_types=[
            pltpu.VMEM((3, CB), jnp.int32),
            pltpu.VMEM((3, CB), jnp.int32),
            pltpu.VMEM((3, CB), jnp.int32),
            pltpu.VMEM((3, CB), jnp.int32),
            pltpu.VMEM((3, CB), jnp.int32),
            pltpu.VMEM((3, CB, HP), jnp.float32),
            pltpu.VMEM((3, CB, HP), jnp.float32),
            pltpu.VMEM((3, CB, 128), jnp.float32),
            pltpu.VMEM_SHARED((NP, 128), jnp.float32),
            pltpu.SemaphoreType.DMA((3,)),
            pltpu.SemaphoreType.DMA((3,)),
            pltpu.SemaphoreType.DMA((3,)),
            pltpu.SemaphoreType.DMA((3,)),
            pltpu.SemaphoreType.DMA((3,)),
        ],
    )
    def k(h_hbm, ex_hbm, rden_hbm, src_hbm, dst_hbm, et_hbm, z_hbm,
          out_hbm,
          src_v, dst_v, et_v, dstc_v, idx_v, ex_v, rd_v, rows_v,
          acc_sp, seml, semr, semd, seme, sems):
        cid = lax.axis_index("c")
        sid = lax.axis_index("s")

        pltpu.sync_copy(z_hbm.at[pl.ds(sid * STRIPE, STRIPE), :],
                        acc_sp.at[pl.ds(sid * STRIPE, STRIPE), :])
        plsc.subcore_barrier()

        NCH = EPW_B // CB
        tbase = sid * EPW_B

        def issue_linear(c, b):
            base = pl.ds(tbase + c * CB, CB)
            pltpu.async_copy(src_hbm.at[base], src_v.at[b], seml.at[b])
            pltpu.async_copy(dst_hbm.at[base], dst_v.at[b], seml.at[b])
            pltpu.async_copy(et_hbm.at[base], et_v.at[b], seml.at[b])

        def wait_linear(c, b):
            base = pl.ds(tbase + c * CB, CB)
            pltpu.make_async_copy(src_hbm.at[base], src_v.at[b],
                                  seml.at[b]).wait()
            pltpu.make_async_copy(dst_hbm.at[base], dst_v.at[b],
                                  seml.at[b]).wait()
            pltpu.make_async_copy(et_hbm.at[base], et_v.at[b],
                                  seml.at[b]).wait()

        def wait_scatter(b):
            pltpu.make_async_copy(rows_v.at[b], acc_sp.at[dstc_v.at[b]],
                                  sems.at[b]).wait()

        def issue_gather(c, b, hh):
            @pl.when(c >= 3)
            def _():
                wait_scatter(b)

            for j in range(CB // 16):
                sl = pl.ds(j * 16, 16)
                dstc_v[b, sl] = dst_v[b, sl]
                idx_v[b, sl] = (et_v[b, sl] * NP + src_v[b, sl]) * 2 + hh
            pltpu.async_copy(h_hbm.at[idx_v.at[b]], rows_v.at[b], semr.at[b])
            pltpu.async_copy(rden_hbm.at[dstc_v.at[b]], rd_v.at[b],
                             semd.at[b])
            pltpu.async_copy(ex_hbm.at[pl.ds(tbase + c * CB, CB), :],
                             ex_v.at[b], seme.at[b])

        def chunk(c, b, hh):
            @pl.when(c + 1 < NCH)
            def _():
                wait_linear(c + 1, (b + 1) % 3)
                issue_gather(c + 1, (b + 1) % 3, hh)

            pltpu.make_async_copy(h_hbm.at[idx_v.at[b]], rows_v.at[b],
                                  semr.at[b]).wait()
            pltpu.make_async_copy(rden_hbm.at[dstc_v.at[b]], rd_v.at[b],
                                  semd.at[b]).wait()
            pltpu.make_async_copy(ex_hbm.at[pl.ds(tbase + c * CB, CB), :],
                                  ex_v.at[b], seme.at[b]).wait()
            h0 = 2 * hh
            for r in range(CB):
                al = ex_v[b, r, :] * rd_v[b, r, :]
                s0 = jnp.broadcast_to(al[h0:h0 + 1], (16,))
                s1 = jnp.broadcast_to(al[h0 + 1:h0 + 2], (16,))
                for v in range(4):
                    sl = pl.ds(v * 16, 16)
                    rows_v[b, r, sl] = rows_v[b, r, sl] * s0
                for v in range(4):
                    sl = pl.ds(64 + v * 16, 16)
                    rows_v[b, r, sl] = rows_v[b, r, sl] * s1
            pltpu.async_copy(rows_v.at[b], acc_sp.at[dstc_v.at[b]],
                             sems.at[b], add=True)

            @pl.when(c + 2 < NCH)
            def _():
                issue_linear(c + 2, (b + 2) % 3)

        def run(hh):
            issue_linear(0, 0)
            wait_linear(0, 0)
            issue_gather(0, 0, hh)
            issue_linear(1, 1)

            def triple(t, carry):
                chunk(3 * t, 0, hh)
                chunk(3 * t + 1, 1, hh)
                chunk(3 * t + 2, 2, hh)
                return carry

            lax.fori_loop(0, NCH // 3, triple, 0)
            for c in range(3 * (NCH // 3), NCH):
                chunk(c, c % 3, hh)
            wait_scatter((NCH - 2) % 3)
            wait_scatter((NCH - 1) % 3)
            wait_scatter(NCH % 3)

        @pl.when(cid == 0)
        def _():
            run(0)

        @pl.when(cid == 1)
        def _():
            run(1)

        plsc.subcore_barrier()
        pltpu.sync_copy(acc_sp.at[pl.ds(sid * STRIPE, STRIPE), :],
                        out_hbm.at[cid, pl.ds(sid * STRIPE, STRIPE), :])

    return k(h_flat, ex, rden, src, dst, et, zeros_d)


def _rden_body(den_ref, out_ref):
    s = den_ref[0] + den_ref[1]
    out_ref[...] = 1.0 / jnp.where(s == 0.0, 1.0, s)


def _den_recip(den):
    BR = 2048
    return pl.pallas_call(
        _rden_body,
        grid=(NP // BR,),
        in_specs=[pl.BlockSpec((2, BR, HP), lambda i: (0, i, 0))],
        out_specs=pl.BlockSpec((BR, HP), lambda i: (i, 0)),
        out_shape=jax.ShapeDtypeStruct((NP, HP), jnp.float32),
    )(den)


def _edge_phase(h_all, aq_node, ak_node, src, dst, et, zeros_h, zeros_d):
    aqf = aq_node.reshape(R * NP, HP)
    akf = ak_node.reshape(R * NP, HP)
    ex, den = _edge_attention_sc(aqf, akf, src, dst, et, zeros_h)
    rden = _den_recip(den)
    h_flat = h_all.reshape(R * NP * 2, 128)
    out = _edge_apply_sc(h_flat, ex, rden, src, dst, et, zeros_d)
    return out


def kernel(x, edge_index, edge_type, batch, W1, q1, k1, b1, W2, q2, k2, b2,
           lin1_w, lin1_b, lin2_w, lin2_b):
    src = edge_index[0]
    dst = edge_index[1]
    et = edge_type

    x_pad = jnp.pad(x, ((0, NP - N), (0, 0)))
    b1_2d = b1.reshape(1, HD)
    b2_2d = b2.reshape(1, D)
    zeros_h = jnp.zeros((NP, HP), jnp.float32)
    zeros_d = jnp.zeros((NP, 128), jnp.float32)

    h1, aq1, ak1 = _relational_transform(x_pad, W1, q1, k1, b1_2d, act=False)
    out1 = _edge_phase(h1, aq1, ak1, src, dst, et, zeros_h, zeros_d)

    h2, aq2, ak2 = _relational_transform2(out1[0], out1[1], W2, q2, k2, b1_2d)
    out2 = _edge_phase(h2, aq2, ak2, src, dst, et, zeros_h, zeros_d)

    batch_pad = jnp.pad(batch, (0, NP - N), constant_values=G)
    batch3 = batch_pad.reshape(NP // BN, 1, BN)
    batch_col = batch_pad.reshape(NP // BN, BN, 1)
    rows = batch_pad.reshape(NP // BN, BN)
    grange = jnp.stack(
        [rows[:, 0], jnp.minimum(rows[:, -1], G - 1)], axis=1
    ).astype(jnp.int32)
    out = pl.pallas_call(
        _pool_body,
        grid=(NP // BN,),
        in_specs=[
            pl.BlockSpec((BN, 128), lambda i: (i, 0)),
            pl.BlockSpec((BN, 128), lambda i: (i, 0)),
            pl.BlockSpec((1, D), lambda i: (0, 0)),
            pl.BlockSpec((1, 1, BN), lambda i: (i, 0, 0)),
            pl.BlockSpec((1, BN, 1), lambda i: (i, 0, 0)),
            pl.BlockSpec(memory_space=pltpu.SMEM),
            pl.BlockSpec((D, D), lambda i: (0, 0)),
            pl.BlockSpec((1, D), lambda i: (0, 0)),
            pl.BlockSpec((D, G), lambda i: (0, 0)),
            pl.BlockSpec((1, G), lambda i: (0, 0)),
        ],
        out_specs=pl.BlockSpec((G, G), lambda i: (0, 0)),
        out_shape=jax.ShapeDtypeStruct((G, G), jnp.float32),
        scratch_shapes=[
            pltpu.VMEM((G, D), jnp.float32),
            pltpu.VMEM((G, D), jnp.float32),
            pltpu.VMEM((G, D), jnp.float32),
        ],
    )(out2[0], out2[1], b2_2d, batch3, batch_col, grange, lin1_w,
      lin1_b.reshape(1, D), lin2_w, lin2_b.reshape(1, G))
    return out


# speedup vs baseline: 40.7878x; 1.0077x over previous
import functools

import jax
import jax.numpy as jnp
from jax import lax
from jax.experimental import pallas as pl
from jax.experimental.pallas import tpu as pltpu
from jax.experimental.pallas import tpu_sc as plsc

N = 10000
E = 320000
R = 8
H = 4
D = 64
G = 64
HD = H * D

NP = 10240
BM = 512
BN = 256
HP = 16


def _mm_body(x_ref, w_ref, q_ref, k_ref, b_ref, h_ref, aq_ref, ak_ref, *, act):
    xb = x_ref[...]
    if act:
        xb = xb + b_ref[...]
        xb = jnp.where(xb > 0, xb, jnp.exp(jnp.minimum(xb, 0.0)) - 1.0)
    h = jnp.dot(xb.astype(jnp.bfloat16), w_ref[0],
                preferred_element_type=jnp.float32)
    h4 = h.reshape(BM, H, D)
    aq = jnp.sum(h4 * q_ref[0][None], axis=-1)
    ak = jnp.sum(h4 * k_ref[0][None], axis=-1)
    zpad = jnp.zeros((BM, HP - H), jnp.float32)
    h_ref[0] = h
    aq_ref[0] = jnp.concatenate([aq, zpad], axis=1)
    ak_ref[0] = jnp.concatenate([ak, zpad], axis=1)


def _relational_transform(x_pad, W, q_att, k_att, bias2d, act):
    F = x_pad.shape[1]
    grid = (R, NP // BM)
    out_shapes = (
        jax.ShapeDtypeStruct((R, NP, HD), jnp.float32),
        jax.ShapeDtypeStruct((R, NP, HP), jnp.float32),
        jax.ShapeDtypeStruct((R, NP, HP), jnp.float32),
    )
    return pl.pallas_call(
        functools.partial(_mm_body, act=act),
        grid=grid,
        in_specs=[
            pl.BlockSpec((BM, F), lambda r, i: (i, 0)),
            pl.BlockSpec((1, F, HD), lambda r, i: (r, 0, 0)),
            pl.BlockSpec((1, H, D), lambda r, i: (r, 0, 0)),
            pl.BlockSpec((1, H, D), lambda r, i: (r, 0, 0)),
            pl.BlockSpec((1, HD), lambda r, i: (0, 0)),
        ],
        out_specs=(
            pl.BlockSpec((1, BM, HD), lambda r, i: (r, i, 0)),
            pl.BlockSpec((1, BM, HP), lambda r, i: (r, i, 0)),
            pl.BlockSpec((1, BM, HP), lambda r, i: (r, i, 0)),
        ),
        out_shape=out_shapes,
    )(x_pad, W.astype(jnp.bfloat16), q_att, k_att, bias2d)


def _mm_body2(ab_ref, w_ref, q_ref, k_ref, bias_ref,
              h_ref, aq_ref, ak_ref):
    bias = bias_ref[...]
    xa = ab_ref[0] + bias[:, 0:128]
    xa = jnp.where(xa > 0, xa, jnp.exp(jnp.minimum(xa, 0.0)) - 1.0)
    xb = ab_ref[1] + bias[:, 128:256]
    xb = jnp.where(xb > 0, xb, jnp.exp(jnp.minimum(xb, 0.0)) - 1.0)
    w = w_ref[0]
    h = (jnp.dot(xa.astype(jnp.bfloat16), w[0:128],
                 preferred_element_type=jnp.float32)
         + jnp.dot(xb.astype(jnp.bfloat16), w[128:256],
                   preferred_element_type=jnp.float32))
    h4 = h.reshape(BM, H, D)
    zpad = jnp.zeros((BM, HP - H), jnp.float32)
    aq_ref[0] = jnp.concatenate(
        [jnp.sum(h4 * q_ref[0][None], axis=-1), zpad], axis=1)
    ak_ref[0] = jnp.concatenate(
        [jnp.sum(h4 * k_ref[0][None], axis=-1), zpad], axis=1)
    h_ref[0] = h


def _relational_transform2(ab, W, q_att, k_att, bias2d):
    grid = (R, NP // BM)
    out_shapes = (
        jax.ShapeDtypeStruct((R, NP, HD), jnp.float32),
        jax.ShapeDtypeStruct((R, NP, HP), jnp.float32),
        jax.ShapeDtypeStruct((R, NP, HP), jnp.float32),
    )
    return pl.pallas_call(
        _mm_body2,
        grid=grid,
        in_specs=[
            pl.BlockSpec((2, BM, 128), lambda r, i: (0, i, 0)),
            pl.BlockSpec((1, HD, HD), lambda r, i: (r, 0, 0)),
            pl.BlockSpec((1, H, D), lambda r, i: (r, 0, 0)),
            pl.BlockSpec((1, H, D), lambda r, i: (r, 0, 0)),
            pl.BlockSpec((1, HD), lambda r, i: (0, 0)),
        ],
        out_specs=(
            pl.BlockSpec((1, BM, HD), lambda r, i: (r, i, 0)),
            pl.BlockSpec((1, BM, HP), lambda r, i: (r, i, 0)),
            pl.BlockSpec((1, BM, HP), lambda r, i: (r, i, 0)),
        ),
        out_shape=out_shapes,
    )(ab, W.astype(jnp.bfloat16), q_att, k_att, bias2d)


def _pool_body(pab_ref, b2_ref, batch_ref, bcol_ref, gr_ref, l1w_ref,
               l1b_ref, l2w_ref, l2b_ref, out_ref, acc_s, acc_c, acc_m):
    i = pl.program_id(0)
    nb = pl.num_programs(0)

    @pl.when(i == 0)
    def _():
        acc_s[...] = jnp.zeros((G, D), jnp.float32)
        acc_c[...] = jnp.zeros((G, D), jnp.float32)
        acc_m[...] = jnp.full((G, D), -jnp.inf, jnp.float32)

    pa = pab_ref[0]
    pb = pab_ref[1]
    zb = (pa[:, 0:64] + pa[:, 64:128] + pb[:, 0:64] + pb[:, 64:128]) * 0.25
    zb = zb + b2_ref[...]
    xb = jnp.where(zb > 0, zb, jnp.exp(jnp.minimum(zb, 0.0)) - 1.0)
    bb = batch_ref[0]
    gids = lax.broadcasted_iota(jnp.int32, (G, BN), 0)
    mask = bb == gids
    maskf = mask.astype(jnp.float32)
    acc_s[...] += jnp.dot(maskf, xb, preferred_element_type=jnp.float32)
    acc_c[...] += jnp.dot(maskf, jnp.ones((BN, D), jnp.float32),
                          preferred_element_type=jnp.float32)
    bcol = bcol_ref[0]
    g_lo = gr_ref[i, 0]
    g_hi = gr_ref[i, 1]

    def _max_body(g, carry):
        m = jnp.where(bcol == g, xb, -jnp.inf)
        mx_g = jnp.max(m, axis=0, keepdims=True)
        acc_m[pl.ds(g, 1), :] = jnp.maximum(acc_m[pl.ds(g, 1), :], mx_g)
        return carry

    lax.fori_loop(g_lo, g_hi + 1, _max_body, 0)

    @pl.when(i == nb - 1)
    def _():
        mean = acc_s[...] / jnp.maximum(acc_c[...], 1.0)
        mxv = acc_m[...]
        mxv = jnp.where(mxv == -jnp.inf, 0.0, mxv)
        pool = mean + mxv
        h = jnp.maximum(
            jnp.dot(pool, l1w_ref[...], preferred_element_type=jnp.float32)
            + l1b_ref[...], 0.0)
        out_ref[...] = (
            jnp.dot(h, l2w_ref[...], preferred_element_type=jnp.float32)
            + l2b_ref[...])


CB = 80
NTILES = 32
EPW_A = E // NTILES
EPW_B = E // 16
STRIPE = NP // 16

_MESH = plsc.VectorSubcoreMesh(core_axis_name="c", subcore_axis_name="s")
_SC_PARAMS = pltpu.CompilerParams(use_tc_tiling_on_sc=False)


def _lanes():
    return lax.iota(jnp.int32, 16)


def _edge_attention_sc(aqf, akf, src, dst, et, zeros_h):
    @functools.partial(
        pl.kernel,
        mesh=_MESH,
        compiler_params=_SC_PARAMS,
        out_type=(
            jax.ShapeDtypeStruct((E, HP), jnp.float32),
            jax.ShapeDtypeStruct((2, NP, HP), jnp.float32),
        ),
        scratch_types=[
            pltpu.VMEM((EPW_A,), jnp.int32),
            pltpu.VMEM((EPW_A,), jnp.int32),
            pltpu.VMEM((EPW_A,), jnp.int32),
            pltpu.VMEM((2, CB), jnp.int32),
            pltpu.VMEM((2, CB), jnp.int32),
            pltpu.VMEM((2, CB), jnp.int32),
            pltpu.VMEM((2, CB, HP), jnp.float32),
            pltpu.VMEM((2, CB, HP), jnp.float32),
            pltpu.VMEM((2, CB, HP), jnp.float32),
            pltpu.VMEM_SHARED((NP, HP), jnp.float32),
            pltpu.SemaphoreType.DMA,
            pltpu.SemaphoreType.DMA,
            pltpu.SemaphoreType.DMA,
            pltpu.SemaphoreType.DMA,
        ],
    )
    def k(aq_hbm, ak_hbm, src_hbm, dst_hbm, et_hbm, z_hbm,
          ex_hbm, den_hbm,
          srcall_v, dstall_v, etall_v, dstc_v, idxq_v, idxk_v,
          aq_v, ak_v, ex_v, den_sp, semq0, semq1, semk0, semk1):
        semq = (semq0, semq1)
        semk = (semk0, semk1)
        cid = lax.axis_index("c")
        sid = lax.axis_index("s")
        wid = sid * 2 + cid

        pltpu.sync_copy(z_hbm.at[pl.ds(sid * STRIPE, STRIPE), :],
                        den_sp.at[pl.ds(sid * STRIPE, STRIPE), :])
        plsc.subcore_barrier()

        NCH = EPW_A // CB
        tbase = wid * EPW_A
        pltpu.sync_copy(src_hbm.at[pl.ds(tbase, EPW_A)], srcall_v)
        pltpu.sync_copy(dst_hbm.at[pl.ds(tbase, EPW_A)], dstall_v)
        pltpu.sync_copy(et_hbm.at[pl.ds(tbase, EPW_A)], etall_v)

        def issue(c, b):
            for j in range(CB // 16):
                sl = pl.ds(c * CB + j * 16, 16)
                so = pl.ds(j * 16, 16)
                etj = etall_v[sl] * NP
                dstj = dstall_v[sl]
                dstc_v[b, so] = dstj
                idxq_v[b, so] = etj + dstj
                idxk_v[b, so] = etj + srcall_v[sl]
            pltpu.async_copy(aq_hbm.at[idxq_v.at[b]], aq_v.at[b], semq[b])
            pltpu.async_copy(ak_hbm.at[idxk_v.at[b]], ak_v.at[b], semk[b])

        issue(0, 0)
        issue(1, 1)

        def chunk(c, b):
            pltpu.make_async_copy(aq_hbm.at[idxq_v.at[b]], aq_v.at[b],
                                  semq[b]).wait()
            pltpu.make_async_copy(ak_hbm.at[idxk_v.at[b]], ak_v.at[b],
                                  semk[b]).wait()
            for r in range(CB):
                ev = aq_v[b, r, :] + ak_v[b, r, :]
                ev = jnp.where(ev >= 0, ev, 0.2 * ev)
                ex_v[b, r, :] = jnp.exp(ev)
            pltpu.sync_copy(ex_v.at[b],
                            ex_hbm.at[pl.ds(tbase + c * CB, CB), :])
            pltpu.sync_copy(ex_v.at[b], den_sp.at[dstc_v.at[b]], add=True)

            @pl.when(c + 2 < NCH)
            def _():
                issue(c + 2, b)

        def pair(t, carry):
            chunk(2 * t, 0)
            chunk(2 * t + 1, 1)
            return carry

        lax.fori_loop(0, NCH // 2, pair, 0)
        if NCH % 2:
            chunk(NCH - 1, 0)
        plsc.subcore_barrier()
        pltpu.sync_copy(den_sp.at[pl.ds(sid * STRIPE, STRIPE), :],
                        den_hbm.at[cid, pl.ds(sid * STRIPE, STRIPE), :])

    return k(aqf, akf, src, dst, et, zeros_h)


def _edge_apply_sc(h_flat, ex, rden, src, dst, et, zeros_d):
    @functools.partial(
        pl.kernel,
        mesh=_MESH,
        compiler_params=_SC_PARAMS,
        out_type=jax.ShapeDtypeStruct((2, NP, 128), jnp.float32),
        scratch_types=[
            pltpu.VMEM((3, CB), jnp.int32),
            pltpu.VMEM((3, CB), jnp.int32),
            pltpu.VMEM((3, CB), jnp.int32),
            pltpu.VMEM((3, CB), jnp.int32),
            pltpu.VMEM((3, CB), jnp.int32),
            pltpu.VMEM((3, CB, HP), jnp.float32),
            pltpu.VMEM((3, CB, HP), jnp.float32),
            pltpu.VMEM((3, CB, 128), jnp.float32),
            pltpu.VMEM_SHARED((NP, 128), jnp.float32),
            pltpu.SemaphoreType.DMA((3,)),
            pltpu.SemaphoreType.DMA((3,)),
            pltpu.SemaphoreType.DMA((3,)),
            pltpu.SemaphoreType.DMA((3,)),
            pltpu.SemaphoreType.DMA((3,)),
        ],
    )
    def k(h_hbm, ex_hbm, rden_hbm, src_hbm, dst_hbm, et_hbm, z_hbm,
          out_hbm,
          src_v, dst_v, et_v, dstc_v, idx_v, ex_v, rd_v, rows_v,
          acc_sp, seml, semr, semd, seme, sems):
        cid = lax.axis_index("c")
        sid = lax.axis_index("s")

        pltpu.sync_copy(z_hbm.at[pl.ds(sid * STRIPE, STRIPE), :],
                        acc_sp.at[pl.ds(sid * STRIPE, STRIPE), :])
        plsc.subcore_barrier()

        NCH = EPW_B // CB
        tbase = sid * EPW_B

        def issue_linear(c, b):
            base = pl.ds(tbase + c * CB, CB)
            pltpu.async_copy(src_hbm.at[base], src_v.at[b], seml.at[b])
            pltpu.async_copy(dst_hbm.at[base], dst_v.at[b], seml.at[b])
            pltpu.async_copy(et_hbm.at[base], et_v.at[b], seml.at[b])

        def wait_linear(c, b):
            base = pl.ds(tbase + c * CB, CB)
            pltpu.make_async_copy(src_hbm.at[base], src_v.at[b],
                                  seml.at[b]).wait()
            pltpu.make_async_copy(dst_hbm.at[base], dst_v.at[b],
                                  seml.at[b]).wait()
            pltpu.make_async_copy(et_hbm.at[base], et_v.at[b],
                                  seml.at[b]).wait()

        def wait_scatter(b):
            pltpu.make_async_copy(rows_v.at[b], acc_sp.at[dstc_v.at[b]],
                                  sems.at[b]).wait()

        def issue_gather(c, b, hh):
            @pl.when(c >= 3)
            def _():
                wait_scatter(b)

            for j in range(CB // 16):
                sl = pl.ds(j * 16, 16)
                dstc_v[b, sl] = dst_v[b, sl]
                idx_v[b, sl] = (et_v[b, sl] * NP + src_v[b, sl]) * 2 + hh
            pltpu.async_copy(h_hbm.at[idx_v.at[b]], rows_v.at[b], semr.at[b])
            pltpu.async_copy(rden_hbm.at[dstc_v.at[b]], rd_v.at[b],
                             semd.at[b])
            pltpu.async_copy(ex_hbm.at[pl.ds(tbase + c * CB, CB), :],
                             ex_v.at[b], seme.at[b])

        def chunk(c, b, hh):
            @pl.when(c + 1 < NCH)
            def _():
                wait_linear(c + 1, (b + 1) % 3)
                issue_gather(c + 1, (b + 1) % 3, hh)

            pltpu.make_async_copy(h_hbm.at[idx_v.at[b]], rows_v.at[b],
                                  semr.at[b]).wait()
            pltpu.make_async_copy(rden_hbm.at[dstc_v.at[b]], rd_v.at[b],
                                  semd.at[b]).wait()
            pltpu.make_async_copy(ex_hbm.at[pl.ds(tbase + c * CB, CB), :],
                                  ex_v.at[b], seme.at[b]).wait()
            h0 = 2 * hh
            for r in range(CB):
                al = ex_v[b, r, :] * rd_v[b, r, :]
                s0 = jnp.broadcast_to(al[h0:h0 + 1], (16,))
                s1 = jnp.broadcast_to(al[h0 + 1:h0 + 2], (16,))
                for v in range(4):
                    sl = pl.ds(v * 16, 16)
                    rows_v[b, r, sl] = rows_v[b, r, sl] * s0
                for v in range(4):
                    sl = pl.ds(64 + v * 16, 16)
                    rows_v[b, r, sl] = rows_v[b, r, sl] * s1
            pltpu.async_copy(rows_v.at[b], acc_sp.at[dstc_v.at[b]],
                             sems.at[b], add=True)

            @pl.when(c + 2 < NCH)
            def _():
                issue_linear(c + 2, (b + 2) % 3)

        def run(hh):
            issue_linear(0, 0)
            wait_linear(0, 0)
            issue_gather(0, 0, hh)
            issue_linear(1, 1)

            def triple(t, carry):
                chunk(3 * t, 0, hh)
                chunk(3 * t + 1, 1, hh)
                chunk(3 * t + 2, 2, hh)
                return carry

            lax.fori_loop(0, NCH // 3, triple, 0)
            for c in range(3 * (NCH // 3), NCH):
                chunk(c, c % 3, hh)
            wait_scatter((NCH - 2) % 3)
            wait_scatter((NCH - 1) % 3)
            wait_scatter(NCH % 3)

        @pl.when(cid == 0)
        def _():
            run(0)

        @pl.when(cid == 1)
        def _():
            run(1)

        plsc.subcore_barrier()
        pltpu.sync_copy(acc_sp.at[pl.ds(sid * STRIPE, STRIPE), :],
                        out_hbm.at[cid, pl.ds(sid * STRIPE, STRIPE), :])

    return k(h_flat, ex, rden, src, dst, et, zeros_d)


def _rden_body(den_ref, out_ref):
    s = den_ref[0] + den_ref[1]
    out_ref[...] = 1.0 / jnp.where(s == 0.0, 1.0, s)


def _den_recip(den):
    BR = 2048
    return pl.pallas_call(
        _rden_body,
        grid=(NP // BR,),
        in_specs=[pl.BlockSpec((2, BR, HP), lambda i: (0, i, 0))],
        out_specs=pl.BlockSpec((BR, HP), lambda i: (i, 0)),
        out_shape=jax.ShapeDtypeStruct((NP, HP), jnp.float32),
    )(den)


def _edge_phase(h_all, aq_node, ak_node, src, dst, et, zeros_h, zeros_d):
    aqf = aq_node.reshape(R * NP, HP)
    akf = ak_node.reshape(R * NP, HP)
    ex, den = _edge_attention_sc(aqf, akf, src, dst, et, zeros_h)
    rden = _den_recip(den)
    h_flat = h_all.reshape(R * NP * 2, 128)
    out = _edge_apply_sc(h_flat, ex, rden, src, dst, et, zeros_d)
    return out


def kernel(x, edge_index, edge_type, batch, W1, q1, k1, b1, W2, q2, k2, b2,
           lin1_w, lin1_b, lin2_w, lin2_b):
    src = edge_index[0]
    dst = edge_index[1]
    et = edge_type

    x_pad = jnp.pad(x, ((0, NP - N), (0, 0)))
    b1_2d = b1.reshape(1, HD)
    b2_2d = b2.reshape(1, D)
    zeros_h = jnp.zeros((NP, HP), jnp.float32)
    zeros_d = jnp.zeros((NP, 128), jnp.float32)

    h1, aq1, ak1 = _relational_transform(x_pad, W1, q1, k1, b1_2d, act=False)
    out1 = _edge_phase(h1, aq1, ak1, src, dst, et, zeros_h, zeros_d)

    h2, aq2, ak2 = _relational_transform2(out1, W2, q2, k2, b1_2d)
    out2 = _edge_phase(h2, aq2, ak2, src, dst, et, zeros_h, zeros_d)

    batch_pad = jnp.pad(batch, (0, NP - N), constant_values=G)
    batch3 = batch_pad.reshape(NP // BN, 1, BN)
    batch_col = batch_pad.reshape(NP // BN, BN, 1)
    rows = batch_pad.reshape(NP // BN, BN)
    grange = jnp.stack(
        [rows[:, 0], jnp.minimum(rows[:, -1], G - 1)], axis=1
    ).astype(jnp.int32)
    out = pl.pallas_call(
        _pool_body,
        grid=(NP // BN,),
        in_specs=[
            pl.BlockSpec((2, BN, 128), lambda i: (0, i, 0)),
            pl.BlockSpec((1, D), lambda i: (0, 0)),
            pl.BlockSpec((1, 1, BN), lambda i: (i, 0, 0)),
            pl.BlockSpec((1, BN, 1), lambda i: (i, 0, 0)),
            pl.BlockSpec(memory_space=pltpu.SMEM),
            pl.BlockSpec((D, D), lambda i: (0, 0)),
            pl.BlockSpec((1, D), lambda i: (0, 0)),
            pl.BlockSpec((D, G), lambda i: (0, 0)),
            pl.BlockSpec((1, G), lambda i: (0, 0)),
        ],
        out_specs=pl.BlockSpec((G, G), lambda i: (0, 0)),
        out_shape=jax.ShapeDtypeStruct((G, G), jnp.float32),
        scratch_shapes=[
            pltpu.VMEM((G, D), jnp.float32),
            pltpu.VMEM((G, D), jnp.float32),
            pltpu.VMEM((G, D), jnp.float32),
        ],
    )(out2, b2_2d, batch3, batch_col, grange, lin1_w,
      lin1_b.reshape(1, D), lin2_w, lin2_b.reshape(1, G))
    return out


# speedup vs baseline: 43.5763x; 1.0684x over previous
import functools

import jax
import jax.numpy as jnp
from jax import lax
from jax.experimental import pallas as pl
from jax.experimental.pallas import tpu as pltpu
from jax.experimental.pallas import tpu_sc as plsc

N = 10000
E = 320000
R = 8
H = 4
D = 64
G = 64
HD = H * D

NP = 10240
BM = 512
BN = 256
HP = 16


def _mm_body(x_ref, w_ref, q_ref, k_ref, b_ref, hlo_ref, hhi_ref, aq_ref,
             ak_ref, *, act):
    xb = x_ref[...]
    if act:
        xb = xb + b_ref[...]
        xb = jnp.where(xb > 0, xb, jnp.exp(jnp.minimum(xb, 0.0)) - 1.0)
    h = jnp.dot(xb.astype(jnp.bfloat16), w_ref[0],
                preferred_element_type=jnp.float32)
    h4 = h.reshape(BM, H, D)
    aq = jnp.sum(h4 * q_ref[0][None], axis=-1)
    ak = jnp.sum(h4 * k_ref[0][None], axis=-1)
    zpad = jnp.zeros((BM, HP - H), jnp.float32)
    hlo_ref[0] = h[:, 0:128]
    hhi_ref[0] = h[:, 128:256]
    aq_ref[0] = jnp.concatenate([aq, zpad], axis=1)
    ak_ref[0] = jnp.concatenate([ak, zpad], axis=1)


def _relational_transform(x_pad, W, q_att, k_att, bias2d, act):
    F = x_pad.shape[1]
    grid = (R, NP // BM)
    out_shapes = (
        jax.ShapeDtypeStruct((R, NP, 128), jnp.float32),
        jax.ShapeDtypeStruct((R, NP, 128), jnp.float32),
        jax.ShapeDtypeStruct((R, NP, HP), jnp.float32),
        jax.ShapeDtypeStruct((R, NP, HP), jnp.float32),
    )
    return pl.pallas_call(
        functools.partial(_mm_body, act=act),
        grid=grid,
        in_specs=[
            pl.BlockSpec((BM, F), lambda r, i: (i, 0)),
            pl.BlockSpec((1, F, HD), lambda r, i: (r, 0, 0)),
            pl.BlockSpec((1, H, D), lambda r, i: (r, 0, 0)),
            pl.BlockSpec((1, H, D), lambda r, i: (r, 0, 0)),
            pl.BlockSpec((1, HD), lambda r, i: (0, 0)),
        ],
        out_specs=(
            pl.BlockSpec((1, BM, 128), lambda r, i: (r, i, 0)),
            pl.BlockSpec((1, BM, 128), lambda r, i: (r, i, 0)),
            pl.BlockSpec((1, BM, HP), lambda r, i: (r, i, 0)),
            pl.BlockSpec((1, BM, HP), lambda r, i: (r, i, 0)),
        ),
        out_shape=out_shapes,
    )(x_pad, W.astype(jnp.bfloat16), q_att, k_att, bias2d)


def _mm_body2(ab_ref, w_ref, q_ref, k_ref, bias_ref,
              hlo_ref, hhi_ref, aq_ref, ak_ref):
    bias = bias_ref[...]
    xa = ab_ref[0] + bias[:, 0:128]
    xa = jnp.where(xa > 0, xa, jnp.exp(jnp.minimum(xa, 0.0)) - 1.0)
    xb = ab_ref[1] + bias[:, 128:256]
    xb = jnp.where(xb > 0, xb, jnp.exp(jnp.minimum(xb, 0.0)) - 1.0)
    w = w_ref[0]
    h = (jnp.dot(xa.astype(jnp.bfloat16), w[0:128],
                 preferred_element_type=jnp.float32)
         + jnp.dot(xb.astype(jnp.bfloat16), w[128:256],
                   preferred_element_type=jnp.float32))
    h4 = h.reshape(BM, H, D)
    zpad = jnp.zeros((BM, HP - H), jnp.float32)
    aq_ref[0] = jnp.concatenate(
        [jnp.sum(h4 * q_ref[0][None], axis=-1), zpad], axis=1)
    ak_ref[0] = jnp.concatenate(
        [jnp.sum(h4 * k_ref[0][None], axis=-1), zpad], axis=1)
    hlo_ref[0] = h[:, 0:128]
    hhi_ref[0] = h[:, 128:256]


def _relational_transform2(ab, W, q_att, k_att, bias2d):
    grid = (R, NP // BM)
    out_shapes = (
        jax.ShapeDtypeStruct((R, NP, 128), jnp.float32),
        jax.ShapeDtypeStruct((R, NP, 128), jnp.float32),
        jax.ShapeDtypeStruct((R, NP, HP), jnp.float32),
        jax.ShapeDtypeStruct((R, NP, HP), jnp.float32),
    )
    return pl.pallas_call(
        _mm_body2,
        grid=grid,
        in_specs=[
            pl.BlockSpec((2, BM, 128), lambda r, i: (0, i, 0)),
            pl.BlockSpec((1, HD, HD), lambda r, i: (r, 0, 0)),
            pl.BlockSpec((1, H, D), lambda r, i: (r, 0, 0)),
            pl.BlockSpec((1, H, D), lambda r, i: (r, 0, 0)),
            pl.BlockSpec((1, HD), lambda r, i: (0, 0)),
        ],
        out_specs=(
            pl.BlockSpec((1, BM, 128), lambda r, i: (r, i, 0)),
            pl.BlockSpec((1, BM, 128), lambda r, i: (r, i, 0)),
            pl.BlockSpec((1, BM, HP), lambda r, i: (r, i, 0)),
            pl.BlockSpec((1, BM, HP), lambda r, i: (r, i, 0)),
        ),
        out_shape=out_shapes,
    )(ab, W.astype(jnp.bfloat16), q_att, k_att, bias2d)


def _pool_body(pab_ref, b2_ref, batch_ref, bcol_ref, gr_ref, l1w_ref,
               l1b_ref, l2w_ref, l2b_ref, out_ref, acc_s, acc_c, acc_m):
    i = pl.program_id(0)
    nb = pl.num_programs(0)

    @pl.when(i == 0)
    def _():
        acc_s[...] = jnp.zeros((G, D), jnp.float32)
        acc_c[...] = jnp.zeros((G, D), jnp.float32)
        acc_m[...] = jnp.full((G, D), -jnp.inf, jnp.float32)

    pa = pab_ref[0]
    pb = pab_ref[1]
    zb = (pa[:, 0:64] + pa[:, 64:128] + pb[:, 0:64] + pb[:, 64:128]) * 0.25
    zb = zb + b2_ref[...]
    xb = jnp.where(zb > 0, zb, jnp.exp(jnp.minimum(zb, 0.0)) - 1.0)
    bb = batch_ref[0]
    gids = lax.broadcasted_iota(jnp.int32, (G, BN), 0)
    mask = bb == gids
    maskf = mask.astype(jnp.float32)
    acc_s[...] += jnp.dot(maskf, xb, preferred_element_type=jnp.float32)
    acc_c[...] += jnp.dot(maskf, jnp.ones((BN, D), jnp.float32),
                          preferred_element_type=jnp.float32)
    bcol = bcol_ref[0]
    g_lo = gr_ref[i, 0]
    g_hi = gr_ref[i, 1]

    def _max_body(g, carry):
        m = jnp.where(bcol == g, xb, -jnp.inf)
        mx_g = jnp.max(m, axis=0, keepdims=True)
        acc_m[pl.ds(g, 1), :] = jnp.maximum(acc_m[pl.ds(g, 1), :], mx_g)
        return carry

    lax.fori_loop(g_lo, g_hi + 1, _max_body, 0)

    @pl.when(i == nb - 1)
    def _():
        mean = acc_s[...] / jnp.maximum(acc_c[...], 1.0)
        mxv = acc_m[...]
        mxv = jnp.where(mxv == -jnp.inf, 0.0, mxv)
        pool = mean + mxv
        h = jnp.maximum(
            jnp.dot(pool, l1w_ref[...], preferred_element_type=jnp.float32)
            + l1b_ref[...], 0.0)
        out_ref[...] = (
            jnp.dot(h, l2w_ref[...], preferred_element_type=jnp.float32)
            + l2b_ref[...])


CB = 80
NTILES = 32
EPW_A = E // NTILES
EPW_B = E // 16
STRIPE = NP // 16

_MESH = plsc.VectorSubcoreMesh(core_axis_name="c", subcore_axis_name="s")
_SC_PARAMS = pltpu.CompilerParams(use_tc_tiling_on_sc=False)


def _lanes():
    return lax.iota(jnp.int32, 16)


def _edge_attention_sc(aqf, akf, src, dst, et, zeros_h):
    @functools.partial(
        pl.kernel,
        mesh=_MESH,
        compiler_params=_SC_PARAMS,
        out_type=(
            jax.ShapeDtypeStruct((E, HP), jnp.float32),
            jax.ShapeDtypeStruct((2, NP, HP), jnp.float32),
        ),
        scratch_types=[
            pltpu.VMEM((EPW_A,), jnp.int32),
            pltpu.VMEM((EPW_A,), jnp.int32),
            pltpu.VMEM((EPW_A,), jnp.int32),
            pltpu.VMEM((2, CB), jnp.int32),
            pltpu.VMEM((2, CB), jnp.int32),
            pltpu.VMEM((2, CB), jnp.int32),
            pltpu.VMEM((2, CB, HP), jnp.float32),
            pltpu.VMEM((2, CB, HP), jnp.float32),
            pltpu.VMEM((2, CB, HP), jnp.float32),
            pltpu.VMEM_SHARED((NP, HP), jnp.float32),
            pltpu.SemaphoreType.DMA,
            pltpu.SemaphoreType.DMA,
            pltpu.SemaphoreType.DMA,
            pltpu.SemaphoreType.DMA,
        ],
    )
    def k(aq_hbm, ak_hbm, src_hbm, dst_hbm, et_hbm, z_hbm,
          ex_hbm, den_hbm,
          srcall_v, dstall_v, etall_v, dstc_v, idxq_v, idxk_v,
          aq_v, ak_v, ex_v, den_sp, semq0, semq1, semk0, semk1):
        semq = (semq0, semq1)
        semk = (semk0, semk1)
        cid = lax.axis_index("c")
        sid = lax.axis_index("s")
        wid = sid * 2 + cid

        pltpu.sync_copy(z_hbm.at[pl.ds(sid * STRIPE, STRIPE), :],
                        den_sp.at[pl.ds(sid * STRIPE, STRIPE), :])
        plsc.subcore_barrier()

        NCH = EPW_A // CB
        tbase = wid * EPW_A
        pltpu.sync_copy(src_hbm.at[pl.ds(tbase, EPW_A)], srcall_v)
        pltpu.sync_copy(dst_hbm.at[pl.ds(tbase, EPW_A)], dstall_v)
        pltpu.sync_copy(et_hbm.at[pl.ds(tbase, EPW_A)], etall_v)

        def issue(c, b):
            for j in range(CB // 16):
                sl = pl.ds(c * CB + j * 16, 16)
                so = pl.ds(j * 16, 16)
                etj = etall_v[sl] * NP
                dstj = dstall_v[sl]
                dstc_v[b, so] = dstj
                idxq_v[b, so] = etj + dstj
                idxk_v[b, so] = etj + srcall_v[sl]
            pltpu.async_copy(aq_hbm.at[idxq_v.at[b]], aq_v.at[b], semq[b])
            pltpu.async_copy(ak_hbm.at[idxk_v.at[b]], ak_v.at[b], semk[b])

        issue(0, 0)
        issue(1, 1)

        def chunk(c, b):
            pltpu.make_async_copy(aq_hbm.at[idxq_v.at[b]], aq_v.at[b],
                                  semq[b]).wait()
            pltpu.make_async_copy(ak_hbm.at[idxk_v.at[b]], ak_v.at[b],
                                  semk[b]).wait()
            for r in range(CB):
                ev = aq_v[b, r, :] + ak_v[b, r, :]
                ev = jnp.where(ev >= 0, ev, 0.2 * ev)
                ex_v[b, r, :] = jnp.exp(ev)
            pltpu.sync_copy(ex_v.at[b],
                            ex_hbm.at[pl.ds(tbase + c * CB, CB), :])
            pltpu.sync_copy(ex_v.at[b], den_sp.at[dstc_v.at[b]], add=True)

            @pl.when(c + 2 < NCH)
            def _():
                issue(c + 2, b)

        def pair(t, carry):
            chunk(2 * t, 0)
            chunk(2 * t + 1, 1)
            return carry

        lax.fori_loop(0, NCH // 2, pair, 0)
        if NCH % 2:
            chunk(NCH - 1, 0)
        plsc.subcore_barrier()
        pltpu.sync_copy(den_sp.at[pl.ds(sid * STRIPE, STRIPE), :],
                        den_hbm.at[cid, pl.ds(sid * STRIPE, STRIPE), :])

    return k(aqf, akf, src, dst, et, zeros_h)


def _edge_apply_sc(h_lo, h_hi, ex, rden, src, dst, et, zeros_d):
    @functools.partial(
        pl.kernel,
        mesh=_MESH,
        compiler_params=_SC_PARAMS,
        out_type=jax.ShapeDtypeStruct((2, NP, 128), jnp.float32),
        scratch_types=[
            pltpu.VMEM((3, CB), jnp.int32),
            pltpu.VMEM((3, CB), jnp.int32),
            pltpu.VMEM((3, CB), jnp.int32),
            pltpu.VMEM((3, CB), jnp.int32),
            pltpu.VMEM((3, CB), jnp.int32),
            pltpu.VMEM((3, CB, HP), jnp.float32),
            pltpu.VMEM((3, CB, HP), jnp.float32),
            pltpu.VMEM((3, CB, 128), jnp.float32),
            pltpu.VMEM_SHARED((NP, 128), jnp.float32),
            pltpu.SemaphoreType.DMA((3,)),
            pltpu.SemaphoreType.DMA((3,)),
            pltpu.SemaphoreType.DMA((3,)),
            pltpu.SemaphoreType.DMA((3,)),
            pltpu.SemaphoreType.DMA((3,)),
        ],
    )
    def k(hlo_hbm, hhi_hbm, ex_hbm, rden_hbm, src_hbm, dst_hbm, et_hbm,
          z_hbm, out_hbm,
          src_v, dst_v, et_v, dstc_v, idx_v, ex_v, rd_v, rows_v,
          acc_sp, seml, semr, semd, seme, sems):
        cid = lax.axis_index("c")
        sid = lax.axis_index("s")

        pltpu.sync_copy(z_hbm.at[pl.ds(sid * STRIPE, STRIPE), :],
                        acc_sp.at[pl.ds(sid * STRIPE, STRIPE), :])
        plsc.subcore_barrier()

        NCH = EPW_B // CB
        tbase = sid * EPW_B

        def issue_linear(c, b):
            base = pl.ds(tbase + c * CB, CB)
            pltpu.async_copy(src_hbm.at[base], src_v.at[b], seml.at[b])
            pltpu.async_copy(dst_hbm.at[base], dst_v.at[b], seml.at[b])
            pltpu.async_copy(et_hbm.at[base], et_v.at[b], seml.at[b])

        def wait_linear(c, b):
            base = pl.ds(tbase + c * CB, CB)
            pltpu.make_async_copy(src_hbm.at[base], src_v.at[b],
                                  seml.at[b]).wait()
            pltpu.make_async_copy(dst_hbm.at[base], dst_v.at[b],
                                  seml.at[b]).wait()
            pltpu.make_async_copy(et_hbm.at[base], et_v.at[b],
                                  seml.at[b]).wait()

        def wait_scatter(b):
            pltpu.make_async_copy(rows_v.at[b], acc_sp.at[dstc_v.at[b]],
                                  sems.at[b]).wait()

        def issue_gather(c, b, hh, h_hbm):
            @pl.when(c >= 3)
            def _():
                wait_scatter(b)

            for j in range(CB // 16):
                sl = pl.ds(j * 16, 16)
                dstc_v[b, sl] = dst_v[b, sl]
                idx_v[b, sl] = et_v[b, sl] * NP + src_v[b, sl]
            pltpu.async_copy(h_hbm.at[idx_v.at[b]], rows_v.at[b], semr.at[b])
            pltpu.async_copy(rden_hbm.at[dstc_v.at[b]], rd_v.at[b],
                             semd.at[b])
            pltpu.async_copy(ex_hbm.at[pl.ds(tbase + c * CB, CB), :],
                             ex_v.at[b], seme.at[b])

        def chunk(c, b, hh, h_hbm):
            @pl.when(c + 1 < NCH)
            def _():
                wait_linear(c + 1, (b + 1) % 3)
                issue_gather(c + 1, (b + 1) % 3, hh, h_hbm)

            pltpu.make_async_copy(h_hbm.at[idx_v.at[b]], rows_v.at[b],
                                  semr.at[b]).wait()
            pltpu.make_async_copy(rden_hbm.at[dstc_v.at[b]], rd_v.at[b],
                                  semd.at[b]).wait()
            pltpu.make_async_copy(ex_hbm.at[pl.ds(tbase + c * CB, CB), :],
                                  ex_v.at[b], seme.at[b]).wait()
            h0 = 2 * hh
            for r in range(CB):
                al = ex_v[b, r, :] * rd_v[b, r, :]
                s0 = jnp.broadcast_to(al[h0:h0 + 1], (16,))
                s1 = jnp.broadcast_to(al[h0 + 1:h0 + 2], (16,))
                for v in range(4):
                    sl = pl.ds(v * 16, 16)
                    rows_v[b, r, sl] = rows_v[b, r, sl] * s0
                for v in range(4):
                    sl = pl.ds(64 + v * 16, 16)
                    rows_v[b, r, sl] = rows_v[b, r, sl] * s1
            pltpu.async_copy(rows_v.at[b], acc_sp.at[dstc_v.at[b]],
                             sems.at[b], add=True)

            @pl.when(c + 2 < NCH)
            def _():
                issue_linear(c + 2, (b + 2) % 3)

        def run(hh, h_hbm):
            issue_linear(0, 0)
            wait_linear(0, 0)
            issue_gather(0, 0, hh, h_hbm)
            issue_linear(1, 1)

            def triple(t, carry):
                chunk(3 * t, 0, hh, h_hbm)
                chunk(3 * t + 1, 1, hh, h_hbm)
                chunk(3 * t + 2, 2, hh, h_hbm)
                return carry

            lax.fori_loop(0, NCH // 3, triple, 0)
            for c in range(3 * (NCH // 3), NCH):
                chunk(c, c % 3, hh, h_hbm)
            wait_scatter((NCH - 2) % 3)
            wait_scatter((NCH - 1) % 3)
            wait_scatter(NCH % 3)

        @pl.when(cid == 0)
        def _():
            run(0, hlo_hbm)

        @pl.when(cid == 1)
        def _():
            run(1, hhi_hbm)

        plsc.subcore_barrier()
        pltpu.sync_copy(acc_sp.at[pl.ds(sid * STRIPE, STRIPE), :],
                        out_hbm.at[cid, pl.ds(sid * STRIPE, STRIPE), :])

    return k(h_lo, h_hi, ex, rden, src, dst, et, zeros_d)


def _rden_body(den_ref, out_ref):
    s = den_ref[0] + den_ref[1]
    out_ref[...] = 1.0 / jnp.where(s == 0.0, 1.0, s)


def _den_recip(den):
    BR = 2048
    return pl.pallas_call(
        _rden_body,
        grid=(NP // BR,),
        in_specs=[pl.BlockSpec((2, BR, HP), lambda i: (0, i, 0))],
        out_specs=pl.BlockSpec((BR, HP), lambda i: (i, 0)),
        out_shape=jax.ShapeDtypeStruct((NP, HP), jnp.float32),
    )(den)


def _edge_phase(h_lo, h_hi, aq_node, ak_node, src, dst, et, zeros_h,
                zeros_d):
    aqf = aq_node.reshape(R * NP, HP)
    akf = ak_node.reshape(R * NP, HP)
    ex, den = _edge_attention_sc(aqf, akf, src, dst, et, zeros_h)
    rden = _den_recip(den)
    out = _edge_apply_sc(h_lo.reshape(R * NP, 128), h_hi.reshape(R * NP, 128),
                         ex, rden, src, dst, et, zeros_d)
    return out


def kernel(x, edge_index, edge_type, batch, W1, q1, k1, b1, W2, q2, k2, b2,
           lin1_w, lin1_b, lin2_w, lin2_b):
    src = edge_index[0]
    dst = edge_index[1]
    et = edge_type

    x_pad = jnp.pad(x, ((0, NP - N), (0, 0)))
    b1_2d = b1.reshape(1, HD)
    b2_2d = b2.reshape(1, D)
    zeros_h = jnp.zeros((NP, HP), jnp.float32)
    zeros_d = jnp.zeros((NP, 128), jnp.float32)

    h1lo, h1hi, aq1, ak1 = _relational_transform(x_pad, W1, q1, k1, b1_2d,
                                                 act=False)
    out1 = _edge_phase(h1lo, h1hi, aq1, ak1, src, dst, et, zeros_h, zeros_d)

    h2lo, h2hi, aq2, ak2 = _relational_transform2(out1, W2, q2, k2, b1_2d)
    out2 = _edge_phase(h2lo, h2hi, aq2, ak2, src, dst, et, zeros_h, zeros_d)

    batch_pad = jnp.pad(batch, (0, NP - N), constant_values=G)
    batch3 = batch_pad.reshape(NP // BN, 1, BN)
    batch_col = batch_pad.reshape(NP // BN, BN, 1)
    rows = batch_pad.reshape(NP // BN, BN)
    grange = jnp.stack(
        [rows[:, 0], jnp.minimum(rows[:, -1], G - 1)], axis=1
    ).astype(jnp.int32)
    out = pl.pallas_call(
        _pool_body,
        grid=(NP // BN,),
        in_specs=[
            pl.BlockSpec((2, BN, 128), lambda i: (0, i, 0)),
            pl.BlockSpec((1, D), lambda i: (0, 0)),
            pl.BlockSpec((1, 1, BN), lambda i: (i, 0, 0)),
            pl.BlockSpec((1, BN, 1), lambda i: (i, 0, 0)),
            pl.BlockSpec(memory_space=pltpu.SMEM),
            pl.BlockSpec((D, D), lambda i: (0, 0)),
            pl.BlockSpec((1, D), lambda i: (0, 0)),
            pl.BlockSpec((D, G), lambda i: (0, 0)),
            pl.BlockSpec((1, G), lambda i: (0, 0)),
        ],
        out_specs=pl.BlockSpec((G, G), lambda i: (0, 0)),
        out_shape=jax.ShapeDtypeStruct((G, G), jnp.float32),
        scratch_shapes=[
            pltpu.VMEM((G, D), jnp.float32),
            pltpu.VMEM((G, D), jnp.float32),
            pltpu.VMEM((G, D), jnp.float32),
        ],
    )(out2, b2_2d, batch3, batch_col, grange, lin1_w,
      lin1_b.reshape(1, D), lin2_w, lin2_b.reshape(1, G))
    return out


# speedup vs baseline: 49.2033x; 1.1291x over previous
import functools

import jax
import jax.numpy as jnp
from jax import lax
from jax.experimental import pallas as pl
from jax.experimental.pallas import tpu as pltpu
from jax.experimental.pallas import tpu_sc as plsc

N = 10000
E = 320000
R = 8
H = 4
D = 64
G = 64
HD = H * D

NP = 10240
BM = 512
BN = 256
HP = 16


def _mm_body(x_ref, w_ref, qk_ref, b_ref, hlo_ref, hhi_ref, aq_ref,
             ak_ref, *, act):
    xb = x_ref[...]
    if act:
        xb = xb + b_ref[...]
        xb = jnp.where(xb > 0, xb, jnp.exp(jnp.minimum(xb, 0.0)) - 1.0)
    h = jnp.dot(xb.astype(jnp.bfloat16), w_ref[0],
                preferred_element_type=jnp.float32)
    aqak = jnp.dot(h.astype(jnp.bfloat16), qk_ref[0],
                   preferred_element_type=jnp.float32)
    zpad = jnp.zeros((BM, HP - H), jnp.float32)
    hlo_ref[...] = h[:, 0:128]
    hhi_ref[...] = h[:, 128:256]
    aq_ref[...] = jnp.concatenate([aqak[:, 0:H], zpad], axis=1)
    ak_ref[...] = jnp.concatenate([aqak[:, H:2 * H], zpad], axis=1)


def _relational_transform(x_pad, W, qk_diag, bias2d, act):
    F = x_pad.shape[1]
    grid = (R, NP // BM)
    NB = NP // BM
    out_shapes = (
        jax.ShapeDtypeStruct((R * NP, 128), jnp.float32),
        jax.ShapeDtypeStruct((R * NP, 128), jnp.float32),
        jax.ShapeDtypeStruct((R * NP, HP), jnp.float32),
        jax.ShapeDtypeStruct((R * NP, HP), jnp.float32),
    )
    return pl.pallas_call(
        functools.partial(_mm_body, act=act),
        grid=grid,
        in_specs=[
            pl.BlockSpec((BM, F), lambda r, i: (i, 0)),
            pl.BlockSpec((1, F, HD), lambda r, i: (r, 0, 0)),
            pl.BlockSpec((1, HD, 2 * H), lambda r, i: (r, 0, 0)),
            pl.BlockSpec((1, HD), lambda r, i: (0, 0)),
        ],
        out_specs=(
            pl.BlockSpec((BM, 128), lambda r, i: (r * NB + i, 0)),
            pl.BlockSpec((BM, 128), lambda r, i: (r * NB + i, 0)),
            pl.BlockSpec((BM, HP), lambda r, i: (r * NB + i, 0)),
            pl.BlockSpec((BM, HP), lambda r, i: (r * NB + i, 0)),
        ),
        out_shape=out_shapes,
    )(x_pad, W.astype(jnp.bfloat16), qk_diag, bias2d)


def _mm_body2(ab_ref, w_ref, qk_ref, bias_ref,
              hlo_ref, hhi_ref, aq_ref, ak_ref):
    bias = bias_ref[...]
    xa = ab_ref[0] + bias[:, 0:128]
    xa = jnp.where(xa > 0, xa, jnp.exp(jnp.minimum(xa, 0.0)) - 1.0)
    xb = ab_ref[1] + bias[:, 128:256]
    xb = jnp.where(xb > 0, xb, jnp.exp(jnp.minimum(xb, 0.0)) - 1.0)
    w = w_ref[0]
    h = (jnp.dot(xa.astype(jnp.bfloat16), w[0:128],
                 preferred_element_type=jnp.float32)
         + jnp.dot(xb.astype(jnp.bfloat16), w[128:256],
                   preferred_element_type=jnp.float32))
    aqak = jnp.dot(h.astype(jnp.bfloat16), qk_ref[0],
                   preferred_element_type=jnp.float32)
    zpad = jnp.zeros((BM, HP - H), jnp.float32)
    aq_ref[...] = jnp.concatenate([aqak[:, 0:H], zpad], axis=1)
    ak_ref[...] = jnp.concatenate([aqak[:, H:2 * H], zpad], axis=1)
    hlo_ref[...] = h[:, 0:128]
    hhi_ref[...] = h[:, 128:256]


def _relational_transform2(ab, W, qk_diag, bias2d):
    grid = (R, NP // BM)
    NB = NP // BM
    out_shapes = (
        jax.ShapeDtypeStruct((R * NP, 128), jnp.float32),
        jax.ShapeDtypeStruct((R * NP, 128), jnp.float32),
        jax.ShapeDtypeStruct((R * NP, HP), jnp.float32),
        jax.ShapeDtypeStruct((R * NP, HP), jnp.float32),
    )
    return pl.pallas_call(
        _mm_body2,
        grid=grid,
        in_specs=[
            pl.BlockSpec((2, BM, 128), lambda r, i: (0, i, 0)),
            pl.BlockSpec((1, HD, HD), lambda r, i: (r, 0, 0)),
            pl.BlockSpec((1, HD, 2 * H), lambda r, i: (r, 0, 0)),
            pl.BlockSpec((1, HD), lambda r, i: (0, 0)),
        ],
        out_specs=(
            pl.BlockSpec((BM, 128), lambda r, i: (r * NB + i, 0)),
            pl.BlockSpec((BM, 128), lambda r, i: (r * NB + i, 0)),
            pl.BlockSpec((BM, HP), lambda r, i: (r * NB + i, 0)),
            pl.BlockSpec((BM, HP), lambda r, i: (r * NB + i, 0)),
        ),
        out_shape=out_shapes,
    )(ab, W.astype(jnp.bfloat16), qk_diag, bias2d)


def _pool_body(pab_ref, b2_ref, batch_ref, bcol_ref, gr_ref, l1w_ref,
               l1b_ref, l2w_ref, l2b_ref, out_ref, acc_s, acc_c, acc_m):
    i = pl.program_id(0)
    nb = pl.num_programs(0)

    @pl.when(i == 0)
    def _():
        acc_s[...] = jnp.zeros((G, D), jnp.float32)
        acc_c[...] = jnp.zeros((G, D), jnp.float32)
        acc_m[...] = jnp.full((G, D), -jnp.inf, jnp.float32)

    pa = pab_ref[0]
    pb = pab_ref[1]
    zb = (pa[:, 0:64] + pa[:, 64:128] + pb[:, 0:64] + pb[:, 64:128]) * 0.25
    zb = zb + b2_ref[...]
    xb = jnp.where(zb > 0, zb, jnp.exp(jnp.minimum(zb, 0.0)) - 1.0)
    bb = batch_ref[0]
    gids = lax.broadcasted_iota(jnp.int32, (G, BN), 0)
    mask = bb == gids
    maskf = mask.astype(jnp.float32)
    acc_s[...] += jnp.dot(maskf, xb, preferred_element_type=jnp.float32)
    acc_c[...] += jnp.dot(maskf, jnp.ones((BN, D), jnp.float32),
                          preferred_element_type=jnp.float32)
    bcol = bcol_ref[0]
    g_lo = gr_ref[i, 0]
    g_hi = gr_ref[i, 1]

    def _max_body(g, carry):
        m = jnp.where(bcol == g, xb, -jnp.inf)
        mx_g = jnp.max(m, axis=0, keepdims=True)
        acc_m[pl.ds(g, 1), :] = jnp.maximum(acc_m[pl.ds(g, 1), :], mx_g)
        return carry

    lax.fori_loop(g_lo, g_hi + 1, _max_body, 0)

    @pl.when(i == nb - 1)
    def _():
        mean = acc_s[...] / jnp.maximum(acc_c[...], 1.0)
        mxv = acc_m[...]
        mxv = jnp.where(mxv == -jnp.inf, 0.0, mxv)
        pool = mean + mxv
        h = jnp.maximum(
            jnp.dot(pool, l1w_ref[...], preferred_element_type=jnp.float32)
            + l1b_ref[...], 0.0)
        out_ref[...] = (
            jnp.dot(h, l2w_ref[...], preferred_element_type=jnp.float32)
            + l2b_ref[...])


CB = 80
NTILES = 32
EPW_A = E // NTILES
EPW_B = E // 16
STRIPE = NP // 16

_MESH = plsc.VectorSubcoreMesh(core_axis_name="c", subcore_axis_name="s")
_SC_PARAMS = pltpu.CompilerParams(use_tc_tiling_on_sc=False)


def _lanes():
    return lax.iota(jnp.int32, 16)


def _edge_attention_sc(aqf, akf, src, dst, et, zeros_h):
    @functools.partial(
        pl.kernel,
        mesh=_MESH,
        compiler_params=_SC_PARAMS,
        out_type=(
            jax.ShapeDtypeStruct((E, HP), jnp.float32),
            jax.ShapeDtypeStruct((2, NP, HP), jnp.float32),
        ),
        scratch_types=[
            pltpu.VMEM((EPW_A,), jnp.int32),
            pltpu.VMEM((EPW_A,), jnp.int32),
            pltpu.VMEM((EPW_A,), jnp.int32),
            pltpu.VMEM((2, CB), jnp.int32),
            pltpu.VMEM((2, CB), jnp.int32),
            pltpu.VMEM((2, CB), jnp.int32),
            pltpu.VMEM((2, CB, HP), jnp.float32),
            pltpu.VMEM((2, CB, HP), jnp.float32),
            pltpu.VMEM((2, CB, HP), jnp.float32),
            pltpu.VMEM_SHARED((NP, HP), jnp.float32),
            pltpu.SemaphoreType.DMA,
            pltpu.SemaphoreType.DMA,
            pltpu.SemaphoreType.DMA,
            pltpu.SemaphoreType.DMA,
        ],
    )
    def k(aq_hbm, ak_hbm, src_hbm, dst_hbm, et_hbm, z_hbm,
          ex_hbm, den_hbm,
          srcall_v, dstall_v, etall_v, dstc_v, idxq_v, idxk_v,
          aq_v, ak_v, ex_v, den_sp, semq0, semq1, semk0, semk1):
        semq = (semq0, semq1)
        semk = (semk0, semk1)
        cid = lax.axis_index("c")
        sid = lax.axis_index("s")
        wid = sid * 2 + cid

        pltpu.sync_copy(z_hbm.at[pl.ds(sid * STRIPE, STRIPE), :],
                        den_sp.at[pl.ds(sid * STRIPE, STRIPE), :])
        plsc.subcore_barrier()

        NCH = EPW_A // CB
        tbase = wid * EPW_A
        pltpu.sync_copy(src_hbm.at[pl.ds(tbase, EPW_A)], srcall_v)
        pltpu.sync_copy(dst_hbm.at[pl.ds(tbase, EPW_A)], dstall_v)
        pltpu.sync_copy(et_hbm.at[pl.ds(tbase, EPW_A)], etall_v)

        def issue(c, b):
            for j in range(CB // 16):
                sl = pl.ds(c * CB + j * 16, 16)
                so = pl.ds(j * 16, 16)
                etj = etall_v[sl] * NP
                dstj = dstall_v[sl]
                dstc_v[b, so] = dstj
                idxq_v[b, so] = etj + dstj
                idxk_v[b, so] = etj + srcall_v[sl]
            pltpu.async_copy(aq_hbm.at[idxq_v.at[b]], aq_v.at[b], semq[b])
            pltpu.async_copy(ak_hbm.at[idxk_v.at[b]], ak_v.at[b], semk[b])

        issue(0, 0)
        issue(1, 1)

        def chunk(c, b):
            pltpu.make_async_copy(aq_hbm.at[idxq_v.at[b]], aq_v.at[b],
                                  semq[b]).wait()
            pltpu.make_async_copy(ak_hbm.at[idxk_v.at[b]], ak_v.at[b],
                                  semk[b]).wait()
            for r in range(CB):
                ev = aq_v[b, r, :] + ak_v[b, r, :]
                ev = jnp.where(ev >= 0, ev, 0.2 * ev)
                ex_v[b, r, :] = jnp.exp(ev)
            pltpu.sync_copy(ex_v.at[b],
                            ex_hbm.at[pl.ds(tbase + c * CB, CB), :])
            pltpu.sync_copy(ex_v.at[b], den_sp.at[dstc_v.at[b]], add=True)

            @pl.when(c + 2 < NCH)
            def _():
                issue(c + 2, b)

        def pair(t, carry):
            chunk(2 * t, 0)
            chunk(2 * t + 1, 1)
            return carry

        lax.fori_loop(0, NCH // 2, pair, 0)
        if NCH % 2:
            chunk(NCH - 1, 0)
        plsc.subcore_barrier()
        pltpu.sync_copy(den_sp.at[pl.ds(sid * STRIPE, STRIPE), :],
                        den_hbm.at[cid, pl.ds(sid * STRIPE, STRIPE), :])

    return k(aqf, akf, src, dst, et, zeros_h)


def _edge_apply_sc(h_lo, h_hi, ex, rden, src, dst, et, zeros_d):
    @functools.partial(
        pl.kernel,
        mesh=_MESH,
        compiler_params=_SC_PARAMS,
        out_type=jax.ShapeDtypeStruct((2, NP, 128), jnp.float32),
        scratch_types=[
            pltpu.VMEM((3, CB), jnp.int32),
            pltpu.VMEM((3, CB), jnp.int32),
            pltpu.VMEM((3, CB), jnp.int32),
            pltpu.VMEM((3, CB), jnp.int32),
            pltpu.VMEM((3, CB), jnp.int32),
            pltpu.VMEM((3, CB, HP), jnp.float32),
            pltpu.VMEM((3, CB, HP), jnp.float32),
            pltpu.VMEM((3, CB, 128), jnp.float32),
            pltpu.VMEM_SHARED((NP, 128), jnp.float32),
            pltpu.SemaphoreType.DMA((3,)),
            pltpu.SemaphoreType.DMA((3,)),
            pltpu.SemaphoreType.DMA((3,)),
            pltpu.SemaphoreType.DMA((3,)),
            pltpu.SemaphoreType.DMA((3,)),
        ],
    )
    def k(hlo_hbm, hhi_hbm, ex_hbm, rden_hbm, src_hbm, dst_hbm, et_hbm,
          z_hbm, out_hbm,
          src_v, dst_v, et_v, dstc_v, idx_v, ex_v, rd_v, rows_v,
          acc_sp, seml, semr, semd, seme, sems):
        cid = lax.axis_index("c")
        sid = lax.axis_index("s")

        pltpu.sync_copy(z_hbm.at[pl.ds(sid * STRIPE, STRIPE), :],
                        acc_sp.at[pl.ds(sid * STRIPE, STRIPE), :])
        plsc.subcore_barrier()

        NCH = EPW_B // CB
        tbase = sid * EPW_B

        def issue_linear(c, b):
            base = pl.ds(tbase + c * CB, CB)
            pltpu.async_copy(src_hbm.at[base], src_v.at[b], seml.at[b])
            pltpu.async_copy(dst_hbm.at[base], dst_v.at[b], seml.at[b])
            pltpu.async_copy(et_hbm.at[base], et_v.at[b], seml.at[b])

        def wait_linear(c, b):
            base = pl.ds(tbase + c * CB, CB)
            pltpu.make_async_copy(src_hbm.at[base], src_v.at[b],
                                  seml.at[b]).wait()
            pltpu.make_async_copy(dst_hbm.at[base], dst_v.at[b],
                                  seml.at[b]).wait()
            pltpu.make_async_copy(et_hbm.at[base], et_v.at[b],
                                  seml.at[b]).wait()

        def wait_scatter(b):
            pltpu.make_async_copy(rows_v.at[b], acc_sp.at[dstc_v.at[b]],
                                  sems.at[b]).wait()

        def issue_gather(c, b, hh, h_hbm):
            @pl.when(c >= 3)
            def _():
                wait_scatter(b)

            for j in range(CB // 16):
                sl = pl.ds(j * 16, 16)
                dstc_v[b, sl] = dst_v[b, sl]
                idx_v[b, sl] = et_v[b, sl] * NP + src_v[b, sl]
            pltpu.async_copy(h_hbm.at[idx_v.at[b]], rows_v.at[b], semr.at[b])
            pltpu.async_copy(rden_hbm.at[dstc_v.at[b]], rd_v.at[b],
                             semd.at[b])
            pltpu.async_copy(ex_hbm.at[pl.ds(tbase + c * CB, CB), :],
                             ex_v.at[b], seme.at[b])

        def chunk(c, b, hh, h_hbm):
            @pl.when(c + 1 < NCH)
            def _():
                wait_linear(c + 1, (b + 1) % 3)
                issue_gather(c + 1, (b + 1) % 3, hh, h_hbm)

            pltpu.make_async_copy(h_hbm.at[idx_v.at[b]], rows_v.at[b],
                                  semr.at[b]).wait()
            pltpu.make_async_copy(rden_hbm.at[dstc_v.at[b]], rd_v.at[b],
                                  semd.at[b]).wait()
            pltpu.make_async_copy(ex_hbm.at[pl.ds(tbase + c * CB, CB), :],
                                  ex_v.at[b], seme.at[b]).wait()
            h0 = 2 * hh
            for r in range(CB):
                al = ex_v[b, r, :] * rd_v[b, r, :]
                s0 = jnp.broadcast_to(al[h0:h0 + 1], (16,))
                s1 = jnp.broadcast_to(al[h0 + 1:h0 + 2], (16,))
                for v in range(4):
                    sl = pl.ds(v * 16, 16)
                    rows_v[b, r, sl] = rows_v[b, r, sl] * s0
                for v in range(4):
                    sl = pl.ds(64 + v * 16, 16)
                    rows_v[b, r, sl] = rows_v[b, r, sl] * s1
            pltpu.async_copy(rows_v.at[b], acc_sp.at[dstc_v.at[b]],
                             sems.at[b], add=True)

            @pl.when(c + 2 < NCH)
            def _():
                issue_linear(c + 2, (b + 2) % 3)

        def run(hh, h_hbm):
            issue_linear(0, 0)
            wait_linear(0, 0)
            issue_gather(0, 0, hh, h_hbm)
            issue_linear(1, 1)

            def triple(t, carry):
                chunk(3 * t, 0, hh, h_hbm)
                chunk(3 * t + 1, 1, hh, h_hbm)
                chunk(3 * t + 2, 2, hh, h_hbm)
                return carry

            lax.fori_loop(0, NCH // 3, triple, 0)
            for c in range(3 * (NCH // 3), NCH):
                chunk(c, c % 3, hh, h_hbm)
            wait_scatter((NCH - 2) % 3)
            wait_scatter((NCH - 1) % 3)
            wait_scatter(NCH % 3)

        @pl.when(cid == 0)
        def _():
            run(0, hlo_hbm)

        @pl.when(cid == 1)
        def _():
            run(1, hhi_hbm)

        plsc.subcore_barrier()
        pltpu.sync_copy(acc_sp.at[pl.ds(sid * STRIPE, STRIPE), :],
                        out_hbm.at[cid, pl.ds(sid * STRIPE, STRIPE), :])

    return k(h_lo, h_hi, ex, rden, src, dst, et, zeros_d)


def _rden_body(den_ref, out_ref):
    s = den_ref[0] + den_ref[1]
    out_ref[...] = 1.0 / jnp.where(s == 0.0, 1.0, s)


def _den_recip(den):
    BR = 2048
    return pl.pallas_call(
        _rden_body,
        grid=(NP // BR,),
        in_specs=[pl.BlockSpec((2, BR, HP), lambda i: (0, i, 0))],
        out_specs=pl.BlockSpec((BR, HP), lambda i: (i, 0)),
        out_shape=jax.ShapeDtypeStruct((NP, HP), jnp.float32),
    )(den)


def _edge_phase(h_lo, h_hi, aqf, akf, src, dst, et, zeros_h, zeros_d):
    ex, den = _edge_attention_sc(aqf, akf, src, dst, et, zeros_h)
    rden = _den_recip(den)
    out = _edge_apply_sc(h_lo, h_hi, ex, rden, src, dst, et, zeros_d)
    return out


def kernel(x, edge_index, edge_type, batch, W1, q1, k1, b1, W2, q2, k2, b2,
           lin1_w, lin1_b, lin2_w, lin2_b):
    src = edge_index[0]
    dst = edge_index[1]
    et = edge_type

    x_pad = jnp.pad(x, ((0, NP - N), (0, 0)))
    hsel = (jnp.arange(HD)[:, None] // D) == jnp.arange(H)[None, :]
    qk1 = jnp.concatenate(
        [jnp.where(hsel, q1.reshape(R, HD, 1), 0.0),
         jnp.where(hsel, k1.reshape(R, HD, 1), 0.0)], axis=2
    ).astype(jnp.bfloat16)
    qk2 = jnp.concatenate(
        [jnp.where(hsel, q2.reshape(R, HD, 1), 0.0),
         jnp.where(hsel, k2.reshape(R, HD, 1), 0.0)], axis=2
    ).astype(jnp.bfloat16)
    b1_2d = b1.reshape(1, HD)
    b2_2d = b2.reshape(1, D)
    zeros_h = jnp.zeros((NP, HP), jnp.float32)
    zeros_d = jnp.zeros((NP, 128), jnp.float32)

    h1lo, h1hi, aq1, ak1 = _relational_transform(x_pad, W1, qk1, b1_2d,
                                                 act=False)
    out1 = _edge_phase(h1lo, h1hi, aq1, ak1, src, dst, et, zeros_h, zeros_d)

    h2lo, h2hi, aq2, ak2 = _relational_transform2(out1, W2, qk2, b1_2d)
    out2 = _edge_phase(h2lo, h2hi, aq2, ak2, src, dst, et, zeros_h, zeros_d)

    batch_pad = jnp.pad(batch, (0, NP - N), constant_values=G)
    batch3 = batch_pad.reshape(NP // BN, 1, BN)
    batch_col = batch_pad.reshape(NP // BN, BN, 1)
    rows = batch_pad.reshape(NP // BN, BN)
    grange = jnp.stack(
        [rows[:, 0], jnp.minimum(rows[:, -1], G - 1)], axis=1
    ).astype(jnp.int32)
    out = pl.pallas_call(
        _pool_body,
        grid=(NP // BN,),
        in_specs=[
            pl.BlockSpec((2, BN, 128), lambda i: (0, i, 0)),
            pl.BlockSpec((1, D), lambda i: (0, 0)),
            pl.BlockSpec((1, 1, BN), lambda i: (i, 0, 0)),
            pl.BlockSpec((1, BN, 1), lambda i: (i, 0, 0)),
            pl.BlockSpec(memory_space=pltpu.SMEM),
            pl.BlockSpec((D, D), lambda i: (0, 0)),
            pl.BlockSpec((1, D), lambda i: (0, 0)),
            pl.BlockSpec((D, G), lambda i: (0, 0)),
            pl.BlockSpec((1, G), lambda i: (0, 0)),
        ],
        out_specs=pl.BlockSpec((G, G), lambda i: (0, 0)),
        out_shape=jax.ShapeDtypeStruct((G, G), jnp.float32),
        scratch_shapes=[
            pltpu.VMEM((G, D), jnp.float32),
            pltpu.VMEM((G, D), jnp.float32),
            pltpu.VMEM((G, D), jnp.float32),
        ],
    )(out2, b2_2d, batch3, batch_col, grange, lin1_w,
      lin1_b.reshape(1, D), lin2_w, lin2_b.reshape(1, G))
    return out


# speedup vs baseline: 50.0293x; 1.0168x over previous
import functools

import jax
import jax.numpy as jnp
from jax import lax
from jax.experimental import pallas as pl
from jax.experimental.pallas import tpu as pltpu
from jax.experimental.pallas import tpu_sc as plsc

N = 10000
E = 320000
R = 8
H = 4
D = 64
G = 64
HD = H * D

NP = 10240
BM = 512
BN = 256
HP = 16


def _mm_body(x_ref, w_ref, qk_ref, b_ref, hlo_ref, hhi_ref, aq_ref,
             ak_ref, *, act):
    xb = x_ref[...]
    if act:
        xb = xb + b_ref[...]
        xb = jnp.where(xb > 0, xb, jnp.exp(jnp.minimum(xb, 0.0)) - 1.0)
    h = jnp.dot(xb.astype(jnp.bfloat16), w_ref[0],
                preferred_element_type=jnp.float32)
    aqak = jnp.dot(h.astype(jnp.bfloat16), qk_ref[0],
                   preferred_element_type=jnp.float32)
    zpad = jnp.zeros((BM, HP - H), jnp.float32)
    hlo_ref[...] = h[:, 0:128]
    hhi_ref[...] = h[:, 128:256]
    aq_ref[...] = jnp.concatenate([aqak[:, 0:H], zpad], axis=1)
    ak_ref[...] = jnp.concatenate([aqak[:, H:2 * H], zpad], axis=1)


def _relational_transform(x_pad, W, qk_diag, bias2d, act):
    F = x_pad.shape[1]
    NB = NP // BM
    grid = (NB, R)
    out_shapes = (
        jax.ShapeDtypeStruct((R * NP, 128), jnp.float32),
        jax.ShapeDtypeStruct((R * NP, 128), jnp.float32),
        jax.ShapeDtypeStruct((R * NP, HP), jnp.float32),
        jax.ShapeDtypeStruct((R * NP, HP), jnp.float32),
    )
    return pl.pallas_call(
        functools.partial(_mm_body, act=act),
        grid=grid,
        in_specs=[
            pl.BlockSpec((BM, F), lambda i, r: (i, 0)),
            pl.BlockSpec((1, F, HD), lambda i, r: (r, 0, 0)),
            pl.BlockSpec((1, HD, 2 * H), lambda i, r: (r, 0, 0)),
            pl.BlockSpec((1, HD), lambda i, r: (0, 0)),
        ],
        out_specs=(
            pl.BlockSpec((BM, 128), lambda i, r: (r * NB + i, 0)),
            pl.BlockSpec((BM, 128), lambda i, r: (r * NB + i, 0)),
            pl.BlockSpec((BM, HP), lambda i, r: (r * NB + i, 0)),
            pl.BlockSpec((BM, HP), lambda i, r: (r * NB + i, 0)),
        ),
        out_shape=out_shapes,
    )(x_pad, W.astype(jnp.bfloat16), qk_diag, bias2d)


def _mm_body2(ab_ref, w_ref, qk_ref, bias_ref,
              hlo_ref, hhi_ref, aq_ref, ak_ref):
    bias = bias_ref[...]
    xa = ab_ref[0] + bias[:, 0:128]
    xa = jnp.where(xa > 0, xa, jnp.exp(jnp.minimum(xa, 0.0)) - 1.0)
    xb = ab_ref[1] + bias[:, 128:256]
    xb = jnp.where(xb > 0, xb, jnp.exp(jnp.minimum(xb, 0.0)) - 1.0)
    w = w_ref[0]
    h = (jnp.dot(xa.astype(jnp.bfloat16), w[0:128],
                 preferred_element_type=jnp.float32)
         + jnp.dot(xb.astype(jnp.bfloat16), w[128:256],
                   preferred_element_type=jnp.float32))
    aqak = jnp.dot(h.astype(jnp.bfloat16), qk_ref[0],
                   preferred_element_type=jnp.float32)
    zpad = jnp.zeros((BM, HP - H), jnp.float32)
    aq_ref[...] = jnp.concatenate([aqak[:, 0:H], zpad], axis=1)
    ak_ref[...] = jnp.concatenate([aqak[:, H:2 * H], zpad], axis=1)
    hlo_ref[...] = h[:, 0:128]
    hhi_ref[...] = h[:, 128:256]


def _relational_transform2(ab, W, qk_diag, bias2d):
    NB = NP // BM
    grid = (NB, R)
    out_shapes = (
        jax.ShapeDtypeStruct((R * NP, 128), jnp.float32),
        jax.ShapeDtypeStruct((R * NP, 128), jnp.float32),
        jax.ShapeDtypeStruct((R * NP, HP), jnp.float32),
        jax.ShapeDtypeStruct((R * NP, HP), jnp.float32),
    )
    return pl.pallas_call(
        _mm_body2,
        grid=grid,
        in_specs=[
            pl.BlockSpec((2, BM, 128), lambda i, r: (0, i, 0)),
            pl.BlockSpec((1, HD, HD), lambda i, r: (r, 0, 0)),
            pl.BlockSpec((1, HD, 2 * H), lambda i, r: (r, 0, 0)),
            pl.BlockSpec((1, HD), lambda i, r: (0, 0)),
        ],
        out_specs=(
            pl.BlockSpec((BM, 128), lambda i, r: (r * NB + i, 0)),
            pl.BlockSpec((BM, 128), lambda i, r: (r * NB + i, 0)),
            pl.BlockSpec((BM, HP), lambda i, r: (r * NB + i, 0)),
            pl.BlockSpec((BM, HP), lambda i, r: (r * NB + i, 0)),
        ),
        out_shape=out_shapes,
    )(ab, W.astype(jnp.bfloat16), qk_diag, bias2d)


def _pool_body(pab_ref, b2_ref, batch_ref, bcol_ref, gr_ref, l1w_ref,
               l1b_ref, l2w_ref, l2b_ref, out_ref, acc_s, acc_c, acc_m):
    i = pl.program_id(0)
    nb = pl.num_programs(0)

    @pl.when(i == 0)
    def _():
        acc_s[...] = jnp.zeros((G, D), jnp.float32)
        acc_c[...] = jnp.zeros((G, D), jnp.float32)
        acc_m[...] = jnp.full((G, D), -jnp.inf, jnp.float32)

    pa = pab_ref[0]
    pb = pab_ref[1]
    zb = (pa[:, 0:64] + pa[:, 64:128] + pb[:, 0:64] + pb[:, 64:128]) * 0.25
    zb = zb + b2_ref[...]
    xb = jnp.where(zb > 0, zb, jnp.exp(jnp.minimum(zb, 0.0)) - 1.0)
    bb = batch_ref[0]
    gids = lax.broadcasted_iota(jnp.int32, (G, BN), 0)
    mask = bb == gids
    maskf = mask.astype(jnp.float32)
    acc_s[...] += jnp.dot(maskf, xb, preferred_element_type=jnp.float32)
    acc_c[...] += jnp.dot(maskf, jnp.ones((BN, D), jnp.float32),
                          preferred_element_type=jnp.float32)
    bcol = bcol_ref[0]
    g_lo = gr_ref[i, 0]
    g_hi = gr_ref[i, 1]

    def _max_body(g, carry):
        m = jnp.where(bcol == g, xb, -jnp.inf)
        mx_g = jnp.max(m, axis=0, keepdims=True)
        acc_m[pl.ds(g, 1), :] = jnp.maximum(acc_m[pl.ds(g, 1), :], mx_g)
        return carry

    lax.fori_loop(g_lo, g_hi + 1, _max_body, 0)

    @pl.when(i == nb - 1)
    def _():
        mean = acc_s[...] / jnp.maximum(acc_c[...], 1.0)
        mxv = acc_m[...]
        mxv = jnp.where(mxv == -jnp.inf, 0.0, mxv)
        pool = mean + mxv
        h = jnp.maximum(
            jnp.dot(pool, l1w_ref[...], preferred_element_type=jnp.float32)
            + l1b_ref[...], 0.0)
        out_ref[...] = (
            jnp.dot(h, l2w_ref[...], preferred_element_type=jnp.float32)
            + l2b_ref[...])


CB = 80
NTILES = 32
EPW_A = E // NTILES
EPW_B = E // 16
STRIPE = NP // 16

_MESH = plsc.VectorSubcoreMesh(core_axis_name="c", subcore_axis_name="s")
_SC_PARAMS = pltpu.CompilerParams(use_tc_tiling_on_sc=False)


def _lanes():
    return lax.iota(jnp.int32, 16)


def _edge_attention_sc(aqf, akf, src, dst, et, zeros_h):
    @functools.partial(
        pl.kernel,
        mesh=_MESH,
        compiler_params=_SC_PARAMS,
        out_type=(
            jax.ShapeDtypeStruct((E, HP), jnp.float32),
            jax.ShapeDtypeStruct((2, NP, HP), jnp.float32),
        ),
        scratch_types=[
            pltpu.VMEM((EPW_A,), jnp.int32),
            pltpu.VMEM((EPW_A,), jnp.int32),
            pltpu.VMEM((EPW_A,), jnp.int32),
            pltpu.VMEM((2, CB), jnp.int32),
            pltpu.VMEM((2, CB), jnp.int32),
            pltpu.VMEM((2, CB), jnp.int32),
            pltpu.VMEM((2, CB, HP), jnp.float32),
            pltpu.VMEM((2, CB, HP), jnp.float32),
            pltpu.VMEM((2, CB, HP), jnp.float32),
            pltpu.VMEM_SHARED((NP, HP), jnp.float32),
            pltpu.SemaphoreType.DMA,
            pltpu.SemaphoreType.DMA,
            pltpu.SemaphoreType.DMA,
            pltpu.SemaphoreType.DMA,
        ],
    )
    def k(aq_hbm, ak_hbm, src_hbm, dst_hbm, et_hbm, z_hbm,
          ex_hbm, den_hbm,
          srcall_v, dstall_v, etall_v, dstc_v, idxq_v, idxk_v,
          aq_v, ak_v, ex_v, den_sp, semq0, semq1, semk0, semk1):
        semq = (semq0, semq1)
        semk = (semk0, semk1)
        cid = lax.axis_index("c")
        sid = lax.axis_index("s")
        wid = sid * 2 + cid

        pltpu.sync_copy(z_hbm.at[pl.ds(sid * STRIPE, STRIPE), :],
                        den_sp.at[pl.ds(sid * STRIPE, STRIPE), :])
        plsc.subcore_barrier()

        NCH = EPW_A // CB
        tbase = wid * EPW_A
        pltpu.sync_copy(src_hbm.at[pl.ds(tbase, EPW_A)], srcall_v)
        pltpu.sync_copy(dst_hbm.at[pl.ds(tbase, EPW_A)], dstall_v)
        pltpu.sync_copy(et_hbm.at[pl.ds(tbase, EPW_A)], etall_v)

        def issue(c, b):
            for j in range(CB // 16):
                sl = pl.ds(c * CB + j * 16, 16)
                so = pl.ds(j * 16, 16)
                etj = etall_v[sl] * NP
                dstj = dstall_v[sl]
                dstc_v[b, so] = dstj
                idxq_v[b, so] = etj + dstj
                idxk_v[b, so] = etj + srcall_v[sl]
            pltpu.async_copy(aq_hbm.at[idxq_v.at[b]], aq_v.at[b], semq[b])
            pltpu.async_copy(ak_hbm.at[idxk_v.at[b]], ak_v.at[b], semk[b])

        issue(0, 0)
        issue(1, 1)

        def chunk(c, b):
            pltpu.make_async_copy(aq_hbm.at[idxq_v.at[b]], aq_v.at[b],
                                  semq[b]).wait()
            pltpu.make_async_copy(ak_hbm.at[idxk_v.at[b]], ak_v.at[b],
                                  semk[b]).wait()
            for r in range(CB):
                ev = aq_v[b, r, :] + ak_v[b, r, :]
                ev = jnp.where(ev >= 0, ev, 0.2 * ev)
                ex_v[b, r, :] = jnp.exp(ev)
            pltpu.sync_copy(ex_v.at[b],
                            ex_hbm.at[pl.ds(tbase + c * CB, CB), :])
            pltpu.sync_copy(ex_v.at[b], den_sp.at[dstc_v.at[b]], add=True)

            @pl.when(c + 2 < NCH)
            def _():
                issue(c + 2, b)

        def pair(t, carry):
            chunk(2 * t, 0)
            chunk(2 * t + 1, 1)
            return carry

        lax.fori_loop(0, NCH // 2, pair, 0)
        if NCH % 2:
            chunk(NCH - 1, 0)
        plsc.subcore_barrier()
        pltpu.sync_copy(den_sp.at[pl.ds(sid * STRIPE, STRIPE), :],
                        den_hbm.at[cid, pl.ds(sid * STRIPE, STRIPE), :])

    return k(aqf, akf, src, dst, et, zeros_h)


def _edge_apply_sc(h_lo, h_hi, ex, rden, src, dst, et, zeros_d):
    @functools.partial(
        pl.kernel,
        mesh=_MESH,
        compiler_params=_SC_PARAMS,
        out_type=jax.ShapeDtypeStruct((2, NP, 128), jnp.float32),
        scratch_types=[
            pltpu.VMEM((3, CB), jnp.int32),
            pltpu.VMEM((3, CB), jnp.int32),
            pltpu.VMEM((3, CB), jnp.int32),
            pltpu.VMEM((3, CB), jnp.int32),
            pltpu.VMEM((3, CB), jnp.int32),
            pltpu.VMEM((3, CB, HP), jnp.float32),
            pltpu.VMEM((3, CB, HP), jnp.float32),
            pltpu.VMEM((3, CB, 128), jnp.float32),
            pltpu.VMEM_SHARED((NP, 128), jnp.float32),
            pltpu.SemaphoreType.DMA((3,)),
            pltpu.SemaphoreType.DMA((3,)),
            pltpu.SemaphoreType.DMA((3,)),
            pltpu.SemaphoreType.DMA((3,)),
            pltpu.SemaphoreType.DMA((3,)),
        ],
    )
    def k(hlo_hbm, hhi_hbm, ex_hbm, rden_hbm, src_hbm, dst_hbm, et_hbm,
          z_hbm, out_hbm,
          src_v, dst_v, et_v, dstc_v, idx_v, ex_v, rd_v, rows_v,
          acc_sp, seml, semr, semd, seme, sems):
        cid = lax.axis_index("c")
        sid = lax.axis_index("s")

        pltpu.sync_copy(z_hbm.at[pl.ds(sid * STRIPE, STRIPE), :],
                        acc_sp.at[pl.ds(sid * STRIPE, STRIPE), :])
        plsc.subcore_barrier()

        NCH = EPW_B // CB
        tbase = sid * EPW_B

        def issue_linear(c, b):
            base = pl.ds(tbase + c * CB, CB)
            pltpu.async_copy(src_hbm.at[base], src_v.at[b], seml.at[b])
            pltpu.async_copy(dst_hbm.at[base], dst_v.at[b], seml.at[b])
            pltpu.async_copy(et_hbm.at[base], et_v.at[b], seml.at[b])

        def wait_linear(c, b):
            base = pl.ds(tbase + c * CB, CB)
            pltpu.make_async_copy(src_hbm.at[base], src_v.at[b],
                                  seml.at[b]).wait()
            pltpu.make_async_copy(dst_hbm.at[base], dst_v.at[b],
                                  seml.at[b]).wait()
            pltpu.make_async_copy(et_hbm.at[base], et_v.at[b],
                                  seml.at[b]).wait()

        def wait_scatter(b):
            pltpu.make_async_copy(rows_v.at[b], acc_sp.at[dstc_v.at[b]],
                                  sems.at[b]).wait()

        def issue_gather(c, b, hh, h_hbm):
            @pl.when(c >= 3)
            def _():
                wait_scatter(b)

            for j in range(CB // 16):
                sl = pl.ds(j * 16, 16)
                dstc_v[b, sl] = dst_v[b, sl]
                idx_v[b, sl] = et_v[b, sl] * NP + src_v[b, sl]
            pltpu.async_copy(h_hbm.at[idx_v.at[b]], rows_v.at[b], semr.at[b])
            pltpu.async_copy(rden_hbm.at[dstc_v.at[b]], rd_v.at[b],
                             semd.at[b])
            pltpu.async_copy(ex_hbm.at[pl.ds(tbase + c * CB, CB), :],
                             ex_v.at[b], seme.at[b])

        def chunk(c, b, hh, h_hbm):
            @pl.when(c + 1 < NCH)
            def _():
                wait_linear(c + 1, (b + 1) % 3)
                issue_gather(c + 1, (b + 1) % 3, hh, h_hbm)

            pltpu.make_async_copy(h_hbm.at[idx_v.at[b]], rows_v.at[b],
                                  semr.at[b]).wait()
            pltpu.make_async_copy(rden_hbm.at[dstc_v.at[b]], rd_v.at[b],
                                  semd.at[b]).wait()
            pltpu.make_async_copy(ex_hbm.at[pl.ds(tbase + c * CB, CB), :],
                                  ex_v.at[b], seme.at[b]).wait()
            h0 = 2 * hh
            for r in range(CB):
                al = ex_v[b, r, :] * rd_v[b, r, :]
                s0 = jnp.broadcast_to(al[h0:h0 + 1], (16,))
                s1 = jnp.broadcast_to(al[h0 + 1:h0 + 2], (16,))
                for v in range(4):
                    sl = pl.ds(v * 16, 16)
                    rows_v[b, r, sl] = rows_v[b, r, sl] * s0
                for v in range(4):
                    sl = pl.ds(64 + v * 16, 16)
                    rows_v[b, r, sl] = rows_v[b, r, sl] * s1
            pltpu.async_copy(rows_v.at[b], acc_sp.at[dstc_v.at[b]],
                             sems.at[b], add=True)

            @pl.when(c + 2 < NCH)
            def _():
                issue_linear(c + 2, (b + 2) % 3)

        def run(hh, h_hbm):
            issue_linear(0, 0)
            wait_linear(0, 0)
            issue_gather(0, 0, hh, h_hbm)
            issue_linear(1, 1)

            def triple(t, carry):
                chunk(3 * t, 0, hh, h_hbm)
                chunk(3 * t + 1, 1, hh, h_hbm)
                chunk(3 * t + 2, 2, hh, h_hbm)
                return carry

            lax.fori_loop(0, NCH // 3, triple, 0)
            for c in range(3 * (NCH // 3), NCH):
                chunk(c, c % 3, hh, h_hbm)
            wait_scatter((NCH - 2) % 3)
            wait_scatter((NCH - 1) % 3)
            wait_scatter(NCH % 3)

        @pl.when(cid == 0)
        def _():
            run(0, hlo_hbm)

        @pl.when(cid == 1)
        def _():
            run(1, hhi_hbm)

        plsc.subcore_barrier()
        pltpu.sync_copy(acc_sp.at[pl.ds(sid * STRIPE, STRIPE), :],
                        out_hbm.at[cid, pl.ds(sid * STRIPE, STRIPE), :])

    return k(h_lo, h_hi, ex, rden, src, dst, et, zeros_d)


def _rden_body(den_ref, out_ref):
    s = den_ref[0] + den_ref[1]
    out_ref[...] = 1.0 / jnp.where(s == 0.0, 1.0, s)


def _den_recip(den):
    BR = 2048
    return pl.pallas_call(
        _rden_body,
        grid=(NP // BR,),
        in_specs=[pl.BlockSpec((2, BR, HP), lambda i: (0, i, 0))],
        out_specs=pl.BlockSpec((BR, HP), lambda i: (i, 0)),
        out_shape=jax.ShapeDtypeStruct((NP, HP), jnp.float32),
    )(den)


def _edge_phase(h_lo, h_hi, aqf, akf, src, dst, et, zeros_h, zeros_d):
    ex, den = _edge_attention_sc(aqf, akf, src, dst, et, zeros_h)
    rden = _den_recip(den)
    out = _edge_apply_sc(h_lo, h_hi, ex, rden, src, dst, et, zeros_d)
    return out


def kernel(x, edge_index, edge_type, batch, W1, q1, k1, b1, W2, q2, k2, b2,
           lin1_w, lin1_b, lin2_w, lin2_b):
    src = edge_index[0]
    dst = edge_index[1]
    et = edge_type

    x_pad = jnp.pad(x, ((0, NP - N), (0, 0)))
    hsel = (jnp.arange(HD)[:, None] // D) == jnp.arange(H)[None, :]
    qk1 = jnp.concatenate(
        [jnp.where(hsel, q1.reshape(R, HD, 1), 0.0),
         jnp.where(hsel, k1.reshape(R, HD, 1), 0.0)], axis=2
    ).astype(jnp.bfloat16)
    qk2 = jnp.concatenate(
        [jnp.where(hsel, q2.reshape(R, HD, 1), 0.0),
         jnp.where(hsel, k2.reshape(R, HD, 1), 0.0)], axis=2
    ).astype(jnp.bfloat16)
    b1_2d = b1.reshape(1, HD)
    b2_2d = b2.reshape(1, D)
    zeros_h = jnp.zeros((NP, HP), jnp.float32)
    zeros_d = jnp.zeros((NP, 128), jnp.float32)

    h1lo, h1hi, aq1, ak1 = _relational_transform(x_pad, W1, qk1, b1_2d,
                                                 act=False)
    out1 = _edge_phase(h1lo, h1hi, aq1, ak1, src, dst, et, zeros_h, zeros_d)

    h2lo, h2hi, aq2, ak2 = _relational_transform2(out1, W2, qk2, b1_2d)
    out2 = _edge_phase(h2lo, h2hi, aq2, ak2, src, dst, et, zeros_h, zeros_d)

    batch_pad = jnp.pad(batch, (0, NP - N), constant_values=G)
    batch3 = batch_pad.reshape(NP // BN, 1, BN)
    batch_col = batch_pad.reshape(NP // BN, BN, 1)
    rows = batch_pad.reshape(NP // BN, BN)
    grange = jnp.stack(
        [rows[:, 0], jnp.minimum(rows[:, -1], G - 1)], axis=1
    ).astype(jnp.int32)
    out = pl.pallas_call(
        _pool_body,
        grid=(NP // BN,),
        in_specs=[
            pl.BlockSpec((2, BN, 128), lambda i: (0, i, 0)),
            pl.BlockSpec((1, D), lambda i: (0, 0)),
            pl.BlockSpec((1, 1, BN), lambda i: (i, 0, 0)),
            pl.BlockSpec((1, BN, 1), lambda i: (i, 0, 0)),
            pl.BlockSpec(memory_space=pltpu.SMEM),
            pl.BlockSpec((D, D), lambda i: (0, 0)),
            pl.BlockSpec((1, D), lambda i: (0, 0)),
            pl.BlockSpec((D, G), lambda i: (0, 0)),
            pl.BlockSpec((1, G), lambda i: (0, 0)),
        ],
        out_specs=pl.BlockSpec((G, G), lambda i: (0, 0)),
        out_shape=jax.ShapeDtypeStruct((G, G), jnp.float32),
        scratch_shapes=[
            pltpu.VMEM((G, D), jnp.float32),
            pltpu.VMEM((G, D), jnp.float32),
            pltpu.VMEM((G, D), jnp.float32),
        ],
    )(out2, b2_2d, batch3, batch_col, grange, lin1_w,
      lin1_b.reshape(1, D), lin2_w, lin2_b.reshape(1, G))
    return out


# speedup vs baseline: 50.1851x; 1.0031x over previous
import functools

import jax
import jax.numpy as jnp
from jax import lax
from jax.experimental import pallas as pl
from jax.experimental.pallas import tpu as pltpu
from jax.experimental.pallas import tpu_sc as plsc

N = 10000
E = 320000
R = 8
H = 4
D = 64
G = 64
HD = H * D

NP = 10240
BM = 512
BN = 256
HP = 16


def _mm_body(x_ref, w_ref, qk_ref, b_ref, hlo_ref, hhi_ref, aq_ref,
             ak_ref, *, act):
    xb = x_ref[...]
    if act:
        xb = xb + b_ref[...]
        xb = jnp.where(xb > 0, xb, jnp.exp(jnp.minimum(xb, 0.0)) - 1.0)
    h = jnp.dot(xb.astype(jnp.bfloat16), w_ref[0],
                preferred_element_type=jnp.float32)
    aqak = jnp.dot(h.astype(jnp.bfloat16), qk_ref[0],
                   preferred_element_type=jnp.float32)
    zpad = jnp.zeros((BM, HP - H), jnp.float32)
    hlo_ref[...] = h[:, 0:128]
    hhi_ref[...] = h[:, 128:256]
    aq_ref[...] = jnp.concatenate([aqak[:, 0:H], zpad], axis=1)
    ak_ref[...] = jnp.concatenate([aqak[:, H:2 * H], zpad], axis=1)


def _relational_transform(x_pad, W, qk_diag, bias2d, act):
    F = x_pad.shape[1]
    NB = NP // BM
    grid = (NB, R)
    out_shapes = (
        jax.ShapeDtypeStruct((R * NP, 128), jnp.float32),
        jax.ShapeDtypeStruct((R * NP, 128), jnp.float32),
        jax.ShapeDtypeStruct((R * NP, HP), jnp.float32),
        jax.ShapeDtypeStruct((R * NP, HP), jnp.float32),
    )
    return pl.pallas_call(
        functools.partial(_mm_body, act=act),
        grid=grid,
        in_specs=[
            pl.BlockSpec((BM, F), lambda i, r: (i, 0)),
            pl.BlockSpec((1, F, HD), lambda i, r: (r, 0, 0)),
            pl.BlockSpec((1, HD, 2 * H), lambda i, r: (r, 0, 0)),
            pl.BlockSpec((1, HD), lambda i, r: (0, 0)),
        ],
        out_specs=(
            pl.BlockSpec((BM, 128), lambda i, r: (r * NB + i, 0)),
            pl.BlockSpec((BM, 128), lambda i, r: (r * NB + i, 0)),
            pl.BlockSpec((BM, HP), lambda i, r: (r * NB + i, 0)),
            pl.BlockSpec((BM, HP), lambda i, r: (r * NB + i, 0)),
        ),
        out_shape=out_shapes,
    )(x_pad, W.astype(jnp.bfloat16), qk_diag, bias2d)


def _mm_body2(ab_ref, w_ref, qk_ref, bias_ref,
              hlo_ref, hhi_ref, aq_ref, ak_ref):
    bias = bias_ref[...]
    xa = ab_ref[0] + bias[:, 0:128]
    xa = jnp.where(xa > 0, xa, jnp.exp(jnp.minimum(xa, 0.0)) - 1.0)
    xb = ab_ref[1] + bias[:, 128:256]
    xb = jnp.where(xb > 0, xb, jnp.exp(jnp.minimum(xb, 0.0)) - 1.0)
    w = w_ref[0]
    h = (jnp.dot(xa.astype(jnp.bfloat16), w[0:128],
                 preferred_element_type=jnp.float32)
         + jnp.dot(xb.astype(jnp.bfloat16), w[128:256],
                   preferred_element_type=jnp.float32))
    aqak = jnp.dot(h.astype(jnp.bfloat16), qk_ref[0],
                   preferred_element_type=jnp.float32)
    zpad = jnp.zeros((BM, HP - H), jnp.float32)
    aq_ref[...] = jnp.concatenate([aqak[:, 0:H], zpad], axis=1)
    ak_ref[...] = jnp.concatenate([aqak[:, H:2 * H], zpad], axis=1)
    hlo_ref[...] = h[:, 0:128]
    hhi_ref[...] = h[:, 128:256]


def _relational_transform2(ab, W, qk_diag, bias2d):
    NB = NP // BM
    grid = (NB, R)
    out_shapes = (
        jax.ShapeDtypeStruct((R * NP, 128), jnp.float32),
        jax.ShapeDtypeStruct((R * NP, 128), jnp.float32),
        jax.ShapeDtypeStruct((R * NP, HP), jnp.float32),
        jax.ShapeDtypeStruct((R * NP, HP), jnp.float32),
    )
    return pl.pallas_call(
        _mm_body2,
        grid=grid,
        in_specs=[
            pl.BlockSpec((2, BM, 128), lambda i, r: (0, i, 0)),
            pl.BlockSpec((1, HD, HD), lambda i, r: (r, 0, 0)),
            pl.BlockSpec((1, HD, 2 * H), lambda i, r: (r, 0, 0)),
            pl.BlockSpec((1, HD), lambda i, r: (0, 0)),
        ],
        out_specs=(
            pl.BlockSpec((BM, 128), lambda i, r: (r * NB + i, 0)),
            pl.BlockSpec((BM, 128), lambda i, r: (r * NB + i, 0)),
            pl.BlockSpec((BM, HP), lambda i, r: (r * NB + i, 0)),
            pl.BlockSpec((BM, HP), lambda i, r: (r * NB + i, 0)),
        ),
        out_shape=out_shapes,
    )(ab, W.astype(jnp.bfloat16), qk_diag, bias2d)


def _pool_body(pab_ref, b2_ref, batch_ref, bcol_ref, gr_ref, l1w_ref,
               l1b_ref, l2w_ref, l2b_ref, out_ref, acc_s, acc_c, acc_m):
    i = pl.program_id(0)
    nb = pl.num_programs(0)

    @pl.when(i == 0)
    def _():
        acc_s[...] = jnp.zeros((G, D), jnp.float32)
        acc_c[...] = jnp.zeros((G, D), jnp.float32)
        acc_m[...] = jnp.full((G, D), -jnp.inf, jnp.float32)

    pa = pab_ref[0]
    pb = pab_ref[1]
    zb = (pa[:, 0:64] + pa[:, 64:128] + pb[:, 0:64] + pb[:, 64:128]) * 0.25
    zb = zb + b2_ref[...]
    xb = jnp.where(zb > 0, zb, jnp.exp(jnp.minimum(zb, 0.0)) - 1.0)
    bb = batch_ref[0]
    gids = lax.broadcasted_iota(jnp.int32, (G, BN), 0)
    mask = bb == gids
    maskf = mask.astype(jnp.float32)
    acc_s[...] += jnp.dot(maskf, xb, preferred_element_type=jnp.float32)
    acc_c[...] += jnp.dot(maskf, jnp.ones((BN, D), jnp.float32),
                          preferred_element_type=jnp.float32)
    bcol = bcol_ref[0]
    g_lo = gr_ref[i, 0]
    g_hi = gr_ref[i, 1]

    def _max_body(g, carry):
        m = jnp.where(bcol == g, xb, -jnp.inf)
        mx_g = jnp.max(m, axis=0, keepdims=True)
        acc_m[pl.ds(g, 1), :] = jnp.maximum(acc_m[pl.ds(g, 1), :], mx_g)
        return carry

    lax.fori_loop(g_lo, g_hi + 1, _max_body, 0)

    @pl.when(i == nb - 1)
    def _():
        mean = acc_s[...] / jnp.maximum(acc_c[...], 1.0)
        mxv = acc_m[...]
        mxv = jnp.where(mxv == -jnp.inf, 0.0, mxv)
        pool = mean + mxv
        h = jnp.maximum(
            jnp.dot(pool, l1w_ref[...], preferred_element_type=jnp.float32)
            + l1b_ref[...], 0.0)
        out_ref[...] = (
            jnp.dot(h, l2w_ref[...], preferred_element_type=jnp.float32)
            + l2b_ref[...])


CB = 80
NTILES = 32
EPW_A = E // NTILES
EPW_B = E // 16
STRIPE = NP // 16

_MESH = plsc.VectorSubcoreMesh(core_axis_name="c", subcore_axis_name="s")
_SC_PARAMS = pltpu.CompilerParams(use_tc_tiling_on_sc=False)


def _edge_attention_sc(aqf, akf, src, dst, et, zeros_h):
    @functools.partial(
        pl.kernel,
        mesh=_MESH,
        compiler_params=_SC_PARAMS,
        out_type=(
            jax.ShapeDtypeStruct((E, HP), jnp.float32),
            jax.ShapeDtypeStruct((2, NP, HP), jnp.float32),
        ),
        scratch_types=[
            pltpu.VMEM((EPW_A,), jnp.int32),
            pltpu.VMEM((EPW_A,), jnp.int32),
            pltpu.VMEM((EPW_A,), jnp.int32),
            pltpu.VMEM((2, CB), jnp.int32),
            pltpu.VMEM((2, CB), jnp.int32),
            pltpu.VMEM((2, CB), jnp.int32),
            pltpu.VMEM((2, CB, HP), jnp.float32),
            pltpu.VMEM((2, CB, HP), jnp.float32),
            pltpu.VMEM((2, CB, HP), jnp.float32),
            pltpu.VMEM_SHARED((NP, HP), jnp.float32),
            pltpu.SemaphoreType.DMA,
            pltpu.SemaphoreType.DMA,
            pltpu.SemaphoreType.DMA,
            pltpu.SemaphoreType.DMA,
        ],
    )
    def k(aq_hbm, ak_hbm, src_hbm, dst_hbm, et_hbm, z_hbm,
          ex_hbm, den_hbm,
          srcall_v, dstall_v, etall_v, dstc_v, idxq_v, idxk_v,
          aq_v, ak_v, ex_v, den_sp, semq0, semq1, semk0, semk1):
        semq = (semq0, semq1)
        semk = (semk0, semk1)
        cid = lax.axis_index("c")
        sid = lax.axis_index("s")
        wid = sid * 2 + cid

        pltpu.sync_copy(z_hbm.at[pl.ds(sid * STRIPE, STRIPE), :],
                        den_sp.at[pl.ds(sid * STRIPE, STRIPE), :])
        plsc.subcore_barrier()

        NCH = EPW_A // CB
        tbase = wid * EPW_A
        pltpu.sync_copy(src_hbm.at[pl.ds(tbase, EPW_A)], srcall_v)
        pltpu.sync_copy(dst_hbm.at[pl.ds(tbase, EPW_A)], dstall_v)
        pltpu.sync_copy(et_hbm.at[pl.ds(tbase, EPW_A)], etall_v)

        def issue(c, b):
            for j in range(CB // 16):
                sl = pl.ds(c * CB + j * 16, 16)
                so = pl.ds(j * 16, 16)
                etj = etall_v[sl] * NP
                dstj = dstall_v[sl]
                dstc_v[b, so] = dstj
                idxq_v[b, so] = etj + dstj
                idxk_v[b, so] = etj + srcall_v[sl]
            pltpu.async_copy(aq_hbm.at[idxq_v.at[b]], aq_v.at[b], semq[b])
            pltpu.async_copy(ak_hbm.at[idxk_v.at[b]], ak_v.at[b], semk[b])

        issue(0, 0)
        issue(1, 1)

        def chunk(c, b):
            pltpu.make_async_copy(aq_hbm.at[idxq_v.at[b]], aq_v.at[b],
                                  semq[b]).wait()
            pltpu.make_async_copy(ak_hbm.at[idxk_v.at[b]], ak_v.at[b],
                                  semk[b]).wait()
            for r in range(CB):
                ev = aq_v[b, r, :] + ak_v[b, r, :]
                ev = jnp.where(ev >= 0, ev, 0.2 * ev)
                ex_v[b, r, :] = jnp.exp(ev)
            pltpu.sync_copy(ex_v.at[b],
                            ex_hbm.at[pl.ds(tbase + c * CB, CB), :])
            pltpu.sync_copy(ex_v.at[b], den_sp.at[dstc_v.at[b]], add=True)

            @pl.when(c + 2 < NCH)
            def _():
                issue(c + 2, b)

        def pair(t, carry):
            chunk(2 * t, 0)
            chunk(2 * t + 1, 1)
            return carry

        lax.fori_loop(0, NCH // 2, pair, 0)
        if NCH % 2:
            chunk(NCH - 1, 0)
        plsc.subcore_barrier()
        pltpu.sync_copy(den_sp.at[pl.ds(sid * STRIPE, STRIPE), :],
                        den_hbm.at[cid, pl.ds(sid * STRIPE, STRIPE), :])

    return k(aqf, akf, src, dst, et, zeros_h)


def _edge_apply_sc(h_lo, h_hi, ex, rden, src, dst, et, zeros_d):
    @functools.partial(
        pl.kernel,
        mesh=_MESH,
        compiler_params=_SC_PARAMS,
        out_type=jax.ShapeDtypeStruct((2, NP, 128), jnp.float32),
        scratch_types=[
            pltpu.VMEM((3, CB), jnp.int32),
            pltpu.VMEM((3, CB), jnp.int32),
            pltpu.VMEM((3, CB), jnp.int32),
            pltpu.VMEM((3, CB), jnp.int32),
            pltpu.VMEM((3, CB), jnp.int32),
            pltpu.VMEM((3, CB, HP), jnp.float32),
            pltpu.VMEM((3, CB, HP), jnp.float32),
            pltpu.VMEM((3, CB, 128), jnp.float32),
            pltpu.VMEM_SHARED((NP, 128), jnp.float32),
            pltpu.SemaphoreType.DMA((3,)),
            pltpu.SemaphoreType.DMA((3,)),
            pltpu.SemaphoreType.DMA((3,)),
            pltpu.SemaphoreType.DMA((3,)),
            pltpu.SemaphoreType.DMA((3,)),
        ],
    )
    def k(hlo_hbm, hhi_hbm, ex_hbm, rden_hbm, src_hbm, dst_hbm, et_hbm,
          z_hbm, out_hbm,
          src_v, dst_v, et_v, dstc_v, idx_v, ex_v, rd_v, rows_v,
          acc_sp, seml, semr, semd, seme, sems):
        cid = lax.axis_index("c")
        sid = lax.axis_index("s")

        pltpu.sync_copy(z_hbm.at[pl.ds(sid * STRIPE, STRIPE), :],
                        acc_sp.at[pl.ds(sid * STRIPE, STRIPE), :])
        plsc.subcore_barrier()

        NCH = EPW_B // CB
        tbase = sid * EPW_B

        def issue_linear(c, b):
            base = pl.ds(tbase + c * CB, CB)
            pltpu.async_copy(src_hbm.at[base], src_v.at[b], seml.at[b])
            pltpu.async_copy(dst_hbm.at[base], dst_v.at[b], seml.at[b])
            pltpu.async_copy(et_hbm.at[base], et_v.at[b], seml.at[b])

        def wait_linear(c, b):
            base = pl.ds(tbase + c * CB, CB)
            pltpu.make_async_copy(src_hbm.at[base], src_v.at[b],
                                  seml.at[b]).wait()
            pltpu.make_async_copy(dst_hbm.at[base], dst_v.at[b],
                                  seml.at[b]).wait()
            pltpu.make_async_copy(et_hbm.at[base], et_v.at[b],
                                  seml.at[b]).wait()

        def wait_scatter(b):
            pltpu.make_async_copy(rows_v.at[b], acc_sp.at[dstc_v.at[b]],
                                  sems.at[b]).wait()

        def issue_gather(c, b, hh, h_hbm):
            @pl.when(c >= 3)
            def _():
                wait_scatter(b)

            for j in range(CB // 16):
                sl = pl.ds(j * 16, 16)
                dstc_v[b, sl] = dst_v[b, sl]
                idx_v[b, sl] = et_v[b, sl] * NP + src_v[b, sl]
            pltpu.async_copy(h_hbm.at[idx_v.at[b]], rows_v.at[b], semr.at[b])
            pltpu.async_copy(rden_hbm.at[dstc_v.at[b]], rd_v.at[b],
                             semd.at[b])
            pltpu.async_copy(ex_hbm.at[pl.ds(tbase + c * CB, CB), :],
                             ex_v.at[b], seme.at[b])

        def chunk(c, b, hh, h_hbm):
            @pl.when(c + 1 < NCH)
            def _():
                wait_linear(c + 1, (b + 1) % 3)
                issue_gather(c + 1, (b + 1) % 3, hh, h_hbm)

            pltpu.make_async_copy(h_hbm.at[idx_v.at[b]], rows_v.at[b],
                                  semr.at[b]).wait()
            pltpu.make_async_copy(rden_hbm.at[dstc_v.at[b]], rd_v.at[b],
                                  semd.at[b]).wait()
            pltpu.make_async_copy(ex_hbm.at[pl.ds(tbase + c * CB, CB), :],
                                  ex_v.at[b], seme.at[b]).wait()
            h0 = 2 * hh
            for r in range(CB):
                al = ex_v[b, r, :] * rd_v[b, r, :]
                s0 = jnp.broadcast_to(al[h0:h0 + 1], (16,))
                s1 = jnp.broadcast_to(al[h0 + 1:h0 + 2], (16,))
                for v in range(4):
                    sl = pl.ds(v * 16, 16)
                    rows_v[b, r, sl] = rows_v[b, r, sl] * s0
                for v in range(4):
                    sl = pl.ds(64 + v * 16, 16)
                    rows_v[b, r, sl] = rows_v[b, r, sl] * s1
            pltpu.async_copy(rows_v.at[b], acc_sp.at[dstc_v.at[b]],
                             sems.at[b], add=True)

            @pl.when(c + 2 < NCH)
            def _():
                issue_linear(c + 2, (b + 2) % 3)

        def run(hh, h_hbm):
            issue_linear(0, 0)
            wait_linear(0, 0)
            issue_gather(0, 0, hh, h_hbm)
            issue_linear(1, 1)

            def triple(t, carry):
                chunk(3 * t, 0, hh, h_hbm)
                chunk(3 * t + 1, 1, hh, h_hbm)
                chunk(3 * t + 2, 2, hh, h_hbm)
                return carry

            lax.fori_loop(0, NCH // 3, triple, 0)
            for c in range(3 * (NCH // 3), NCH):
                chunk(c, c % 3, hh, h_hbm)
            wait_scatter((NCH - 2) % 3)
            wait_scatter((NCH - 1) % 3)
            wait_scatter(NCH % 3)

        @pl.when(cid == 0)
        def _():
            run(0, hlo_hbm)

        @pl.when(cid == 1)
        def _():
            run(1, hhi_hbm)

        plsc.subcore_barrier()
        pltpu.sync_copy(acc_sp.at[pl.ds(sid * STRIPE, STRIPE), :],
                        out_hbm.at[cid, pl.ds(sid * STRIPE, STRIPE), :])

    return k(h_lo, h_hi, ex, rden, src, dst, et, zeros_d)


def _rden_body(den_ref, out_ref):
    s = den_ref[0] + den_ref[1]
    out_ref[...] = 1.0 / jnp.where(s == 0.0, 1.0, s)


def _den_recip(den):
    BR = 2048
    return pl.pallas_call(
        _rden_body,
        grid=(NP // BR,),
        in_specs=[pl.BlockSpec((2, BR, HP), lambda i: (0, i, 0))],
        out_specs=pl.BlockSpec((BR, HP), lambda i: (i, 0)),
        out_shape=jax.ShapeDtypeStruct((NP, HP), jnp.float32),
    )(den)


def _edge_phase(h_lo, h_hi, aqf, akf, src, dst, et, zeros_h, zeros_d):
    ex, den = _edge_attention_sc(aqf, akf, src, dst, et, zeros_h)
    rden = _den_recip(den)
    out = _edge_apply_sc(h_lo, h_hi, ex, rden, src, dst, et, zeros_d)
    return out


def kernel(x, edge_index, edge_type, batch, W1, q1, k1, b1, W2, q2, k2, b2,
           lin1_w, lin1_b, lin2_w, lin2_b):
    src = edge_index[0]
    dst = edge_index[1]
    et = edge_type

    x_pad = jnp.pad(x, ((0, NP - N), (0, 0)))
    hsel = (jnp.arange(HD)[:, None] // D) == jnp.arange(H)[None, :]
    qk1 = jnp.concatenate(
        [jnp.where(hsel, q1.reshape(R, HD, 1), 0.0),
         jnp.where(hsel, k1.reshape(R, HD, 1), 0.0)], axis=2
    ).astype(jnp.bfloat16)
    qk2 = jnp.concatenate(
        [jnp.where(hsel, q2.reshape(R, HD, 1), 0.0),
         jnp.where(hsel, k2.reshape(R, HD, 1), 0.0)], axis=2
    ).astype(jnp.bfloat16)
    b1_2d = b1.reshape(1, HD)
    b2_2d = b2.reshape(1, D)
    zeros_h = jnp.zeros((NP, HP), jnp.float32)
    zeros_d = jnp.zeros((NP, 128), jnp.float32)

    h1lo, h1hi, aq1, ak1 = _relational_transform(x_pad, W1, qk1, b1_2d,
                                                 act=False)
    out1 = _edge_phase(h1lo, h1hi, aq1, ak1, src, dst, et, zeros_h, zeros_d)

    h2lo, h2hi, aq2, ak2 = _relational_transform2(out1, W2, qk2, b1_2d)
    out2 = _edge_phase(h2lo, h2hi, aq2, ak2, src, dst, et, zeros_h, zeros_d)

    batch_pad = jnp.pad(batch, (0, NP - N), constant_values=G)
    batch3 = batch_pad.reshape(NP // BN, 1, BN)
    batch_col = batch_pad.reshape(NP // BN, BN, 1)
    rows = batch_pad.reshape(NP // BN, BN)
    grange = jnp.stack(
        [rows[:, 0], jnp.minimum(rows[:, -1], G - 1)], axis=1
    ).astype(jnp.int32)
    out = pl.pallas_call(
        _pool_body,
        grid=(NP // BN,),
        in_specs=[
            pl.BlockSpec((2, BN, 128), lambda i: (0, i, 0)),
            pl.BlockSpec((1, D), lambda i: (0, 0)),
            pl.BlockSpec((1, 1, BN), lambda i: (i, 0, 0)),
            pl.BlockSpec((1, BN, 1), lambda i: (i, 0, 0)),
            pl.BlockSpec(memory_space=pltpu.SMEM),
            pl.BlockSpec((D, D), lambda i: (0, 0)),
            pl.BlockSpec((1, D), lambda i: (0, 0)),
            pl.BlockSpec((D, G), lambda i: (0, 0)),
            pl.BlockSpec((1, G), lambda i: (0, 0)),
        ],
        out_specs=pl.BlockSpec((G, G), lambda i: (0, 0)),
        out_shape=jax.ShapeDtypeStruct((G, G), jnp.float32),
        scratch_shapes=[
            pltpu.VMEM((G, D), jnp.float32),
            pltpu.VMEM((G, D), jnp.float32),
            pltpu.VMEM((G, D), jnp.float32),
        ],
    )(out2, b2_2d, batch3, batch_col, grange, lin1_w,
      lin1_b.reshape(1, D), lin2_w, lin2_b.reshape(1, G))
    return out
